# Initial kernel scaffold; baseline (speedup 1.0000x reference)
import jax
import jax.numpy as jnp
from jax.experimental import pallas as pl


def kernel(x, edge_index, edge_attr, W_f, b_f, W_q, b_q, W_k, b_k, W_v, b_v, W_e, W_skip, b_skip):
    raise NotImplementedError("write your pallas kernel here")


# speedup vs baseline: 1.2086x; 1.2086x over previous
import functools

import jax
import jax.numpy as jnp
from jax.experimental import pallas as pl

N = 10000
E = 320000
D = 128
C = 128


def _mm_kernel(h_ref, w_ref, b_ref, o_ref):
    o_ref[...] = h_ref[...] @ w_ref[...] + b_ref[...]


def _dense(h, W, b):
    return pl.pallas_call(
        _mm_kernel,
        out_shape=jax.ShapeDtypeStruct((h.shape[0], W.shape[1]), jnp.float32),
        grid=(10,),
        in_specs=[
            pl.BlockSpec((h.shape[0] // 10, h.shape[1]), lambda i: (i, 0)),
            pl.BlockSpec((W.shape[0], W.shape[1]), lambda i: (0, 0)),
            pl.BlockSpec((W.shape[1],), lambda i: (0,)),
        ],
        out_specs=pl.BlockSpec((h.shape[0] // 10, W.shape[1]), lambda i: (i, 0)),
    )(h, W, b)


def _conv(h, src, dst, edge_attr, W_q, b_q, W_k, b_k, W_v, b_v, W_e, W_skip, b_skip):
    q = _dense(h, W_q, b_q)
    k = _dense(h, W_k, b_k)
    v = _dense(h, W_v, b_v)
    qe = q @ W_e.T
    inv = 1.0 / jnp.sqrt(float(C))
    alpha = (jnp.einsum("ec,ec->e", q[dst], k[src]) +
             jnp.einsum("ed,ed->e", qe[dst], edge_attr)) * inv
    amax = jax.ops.segment_max(alpha, dst, num_segments=N)
    amax = jnp.where(jnp.isfinite(amax), amax, 0.0)
    ex = jnp.exp(alpha - amax[dst])
    denom = jax.ops.segment_sum(ex, dst, num_segments=N)
    acc = jax.ops.segment_sum(v[src] * ex[:, None], dst, num_segments=N)
    s = jax.ops.segment_sum(edge_attr * ex[:, None], dst, num_segments=N)
    out = (acc + s @ W_e) / (denom[:, None] + 1e-16)
    out = out + _dense(h, W_skip, b_skip)
    return out


def kernel(x, edge_index, edge_attr, W_f, b_f, W_q, b_q, W_k, b_k, W_v, b_v, W_e, W_skip, b_skip):
    src = edge_index[0]
    dst = edge_index[1]
    h = jax.nn.leaky_relu(_dense(x, W_f, b_f))
    for _ in range(3):
        h = _conv(h, src, dst, edge_attr, W_q, b_q, W_k, b_k, W_v, b_v, W_e, W_skip, b_skip)
        h = jax.nn.leaky_relu(h)
    return h


# speedup vs baseline: 3.1745x; 2.6266x over previous
import functools

import jax
import jax.numpy as jnp
from jax import lax
from jax.experimental import pallas as pl
from jax.experimental.pallas import tpu as pltpu
from jax.experimental.pallas import tpu_sc as plsc

NN = 10000
EE = 320000
CC = 128
NPAD = 10240
NC, NS = 2, 16
NW = NC * NS
EPW = EE // NW
CB = 80
NG = CB // 16
NCHUNK = EPW // CB
SLICE = NPAD // NS
ACCW = 144

_MESH = plsc.VectorSubcoreMesh(core_axis_name="c", subcore_axis_name="s",
                               num_cores=NC, num_subcores=NS)

_LANE = lambda: lax.iota(jnp.int32, 16)
_GDN = lax.GatherDimensionNumbers(offset_dims=(), collapsed_slice_dims=(0,),
                                  start_index_map=(0,))


def _take16(x, i):
    return lax.gather(x, i[:, None], _GDN, (1,),
                      mode=lax.GatherScatterMode.PROMISE_IN_BOUNDS)


def _seg_max_update(amax_ref, d16, a16):
    lane = _LANE()
    k16, v16 = plsc.sort_key_val(d16, a16)
    for sh in (1, 2, 4, 8):
        idx = jnp.maximum(lane - sh, 0)
        pk = _take16(k16, idx)
        pv = _take16(v16, idx)
        v16 = jnp.where((pk == k16) & (lane >= sh), jnp.maximum(v16, pv), v16)
    nxt = _take16(k16, jnp.minimum(lane + 1, 15))
    last = (lane == 15) | (k16 != nxt)
    old = plsc.load_gather(amax_ref, [k16])
    plsc.store_scatter(amax_ref, [k16], jnp.maximum(old, v16), mask=last)


def _passA_body(qs_hbm, k_hbm, qeT_hbm, ei_hbm, ea_hbm,
                alpha_hbm, amax2_hbm,
                qe0, qe1, amax, dst_i, src_i, ea_c, qrows, krows,
                alpha_c, red, stage, sem1, sem2):
    c = lax.axis_index("c")
    s = lax.axis_index("s")
    wid = c * NS + s
    ebase = wid * EPW
    lane = _LANE()
    zero = jnp.zeros((16,), jnp.float32)
    zi = jnp.zeros((16,), jnp.int32)

    pltpu.sync_copy(qeT_hbm.at[pl.ds(0, NN)], qe0)
    pltpu.sync_copy(qeT_hbm.at[pl.ds(NN, NN)], qe1)

    neg = jnp.full((16,), -3.0e38, jnp.float32)

    def init_body(i, carry):
        amax[pl.ds(i * 16, 16)] = neg
        return carry

    lax.fori_loop(0, NPAD // 16, init_body, 0)

    def chunk_body(ci, carry):
        base = ebase + ci * CB
        pltpu.sync_copy(ei_hbm.at[pl.ds(EE + base, CB)], dst_i)
        pltpu.sync_copy(ei_hbm.at[pl.ds(base, CB)], src_i)
        pltpu.sync_copy(ea_hbm.at[pl.ds(base, CB), :], ea_c)
        d1 = pltpu.async_copy(qs_hbm.at[dst_i], qrows, sem1)
        d2 = pltpu.async_copy(k_hbm.at[src_i], krows, sem2)
        d1.wait()
        d2.wait()

        def group_body(g, carry2):
            def edge_body(e, avec):
                ea_ = g * 16 + e
                acc = qrows[ea_, pl.ds(0, 16)] * krows[ea_, pl.ds(0, 16)]
                for r in range(1, 8):
                    acc = acc + (qrows[ea_, pl.ds(r * 16, 16)] *
                                 krows[ea_, pl.ds(r * 16, 16)])
                dot = jnp.sum(acc)
                return jnp.where(lane == e, jnp.full((16,), dot, jnp.float32),
                                 avec)

            avec = lax.fori_loop(0, 16, edge_body, zero)
            d16 = dst_i[pl.ds(g * 16, 16)]
            e16 = g * 16 + lane
            ea0 = plsc.load_gather(ea_c, [e16, zi])
            ea1 = plsc.load_gather(ea_c, [e16, zi + 1])
            q0 = plsc.load_gather(qe0, [d16])
            q1 = plsc.load_gather(qe1, [d16])
            a16 = avec + q0 * ea0 + q1 * ea1
            alpha_c[pl.ds(g * 16, 16)] = a16
            _seg_max_update(amax, d16, a16)
            return carry2

        lax.fori_loop(0, NG, group_body, 0)
        pltpu.sync_copy(alpha_c, alpha_hbm.at[pl.ds(base, CB)])
        return carry

    lax.fori_loop(0, NCHUNK, chunk_body, 0)

    pltpu.sync_copy(amax, stage.at[pl.ds(s * NPAD, NPAD)])
    plsc.subcore_barrier()
    for j in range(NS):
        pltpu.sync_copy(stage.at[pl.ds(j * NPAD + s * SLICE, SLICE)],
                        red.at[pl.ds(j * SLICE, SLICE)])

    def red_body(i, carry):
        m = red[pl.ds(i * 16, 16)]
        for j in range(1, NS):
            m = jnp.maximum(m, red[pl.ds(j * SLICE + i * 16, 16)])
        amax[pl.ds(s * SLICE + i * 16, 16)] = m
        return carry

    lax.fori_loop(0, SLICE // 16, red_body, 0)
    pltpu.sync_copy(amax.at[pl.ds(s * SLICE, SLICE)],
                    amax2_hbm.at[pl.ds(c * NPAD + s * SLICE, SLICE)])


_passA = pl.kernel(
    _passA_body,
    out_type=(
        jax.ShapeDtypeStruct((EE,), jnp.float32),
        jax.ShapeDtypeStruct((NC * NPAD,), jnp.float32),
    ),
    mesh=_MESH,
    compiler_params=pltpu.CompilerParams(needs_layout_passes=False),
    scratch_types=[
        pltpu.VMEM((NN,), jnp.float32),
        pltpu.VMEM((NN,), jnp.float32),
        pltpu.VMEM((NPAD,), jnp.float32),
        pltpu.VMEM((CB,), jnp.int32),
        pltpu.VMEM((CB,), jnp.int32),
        pltpu.VMEM((CB, 2), jnp.float32),
        pltpu.VMEM((CB, CC), jnp.float32),
        pltpu.VMEM((CB, CC), jnp.float32),
        pltpu.VMEM((CB,), jnp.float32),
        pltpu.VMEM((NS * SLICE,), jnp.float32),
        pltpu.VMEM_SHARED((NS * NPAD,), jnp.float32),
        pltpu.SemaphoreType.DMA,
        pltpu.SemaphoreType.DMA,
    ],
)


def _seg_sum_update(meta_ref, d16, vals, offs):
    lane = _LANE()
    k16, perm = plsc.sort_key_val(d16, lane)
    vs = [_take16(v, perm) for v in vals]
    pks = []
    for sh in (1, 2, 4, 8):
        idx = jnp.maximum(lane - sh, 0)
        pk = _take16(k16, idx)
        m = (pk == k16) & (lane >= sh)
        vs = [v + jnp.where(m, _take16(v, idx), 0.0) for v in vs]
    nxt = _take16(k16, jnp.minimum(lane + 1, 15))
    last = (lane == 15) | (k16 != nxt)
    for v, off in zip(vs, offs):
        plsc.addupdate_scatter(meta_ref, [k16 + off], v, mask=last)


EPWB = EE // NS
NCHUNKB = EPWB // CB
NHALF = NPAD // 2
SLH = NHALF // NS
NACC = 5184


def _passB_body(v_hbm, ei_hbm, ea_hbm, alpha_hbm, amax2_hbm,
                accout_hbm, meta2_hbm,
                amax, tmpa, meta, dst_i, src_i, sidx, ea_c, alpha_c,
                vrows, buf, acc, sem1):
    c = lax.axis_index("c")
    s = lax.axis_index("s")
    wid = c * NS + s
    ebase = s * EPWB
    lane = _LANE()
    zero = jnp.zeros((16,), jnp.float32)
    zi = jnp.zeros((16,), jnp.int32)
    cbase = c * NHALF
    trash = NHALF + s

    pltpu.sync_copy(amax2_hbm.at[pl.ds(0, NPAD)], amax)
    pltpu.sync_copy(amax2_hbm.at[pl.ds(NPAD, NPAD)], tmpa)

    def maxb(i, carry):
        amax[pl.ds(i * 16, 16)] = jnp.maximum(amax[pl.ds(i * 16, 16)],
                                              tmpa[pl.ds(i * 16, 16)])
        return carry

    lax.fori_loop(0, NPAD // 16, maxb, 0)

    def zm(i, carry):
        meta[pl.ds(i * 16, 16)] = zero
        return carry

    lax.fori_loop(0, 3 * NPAD // 16, zm, 0)

    def zb(e, carry):
        for r in range(CC // 16):
            buf[e, pl.ds(r * 16, 16)] = zero
        return carry

    lax.fori_loop(0, CB, zb, 0)
    for j in range(SLH // CB):
        pltpu.sync_copy(buf, acc.at[pl.ds(s * SLH + j * CB, CB), :])
    pl.when(s == 0)(lambda: pltpu.sync_copy(
        buf.at[pl.ds(0, NACC - NHALF), :], acc.at[pl.ds(NHALF, NACC - NHALF), :]))
    plsc.subcore_barrier()

    def chunk_body(ci, carry):
        base = ebase + ci * CB
        pltpu.sync_copy(ei_hbm.at[pl.ds(EE + base, CB)], dst_i)
        pltpu.sync_copy(ei_hbm.at[pl.ds(base, CB)], src_i)
        pltpu.sync_copy(ea_hbm.at[pl.ds(base, CB), :], ea_c)
        pltpu.sync_copy(alpha_hbm.at[pl.ds(base, CB)], alpha_c)
        d1 = pltpu.async_copy(v_hbm.at[src_i], vrows, sem1)

        def remap(g, carry2):
            d16 = dst_i[pl.ds(g * 16, 16)]
            inr = (d16 >= cbase) & (d16 < cbase + NHALF)
            sidx[pl.ds(g * 16, 16)] = jnp.where(inr, d16 - cbase, trash)
            return carry2

        lax.fori_loop(0, NG, remap, 0)
        d1.wait()

        def group_body(g, carry2):
            d16 = dst_i[pl.ds(g * 16, 16)]
            m16 = plsc.load_gather(amax, [d16])
            a16 = alpha_c[pl.ds(g * 16, 16)]
            ex16 = jnp.exp(a16 - m16)
            e16 = g * 16 + lane
            ea0 = plsc.load_gather(ea_c, [e16, zi])
            ea1 = plsc.load_gather(ea_c, [e16, zi + 1])
            _seg_sum_update(meta, d16, [ex16, ex16 * ea0, ex16 * ea1],
                            [0, NPAD, 2 * NPAD])

            def edge_body(e, carry3):
                ea_ = g * 16 + e
                xs = jnp.sum(jnp.where(lane == e, ex16, zero))
                for r in range(CC // 16):
                    buf[ea_, pl.ds(r * 16, 16)] = vrows[ea_, pl.ds(r * 16, 16)] * xs
                return carry3

            lax.fori_loop(0, 16, edge_body, 0)
            return carry2

        lax.fori_loop(0, NG, group_body, 0)
        pltpu.sync_copy(buf, acc.at[sidx], add=True)
        return carry

    lax.fori_loop(0, NCHUNKB, chunk_body, 0)

    pltpu.sync_copy(meta, meta2_hbm.at[pl.ds(wid * 3 * NPAD, 3 * NPAD)])
    plsc.subcore_barrier()
    pltpu.sync_copy(acc.at[pl.ds(s * SLH, SLH), :],
                    accout_hbm.at[pl.ds(c * NHALF + s * SLH, SLH), :])


_passB = pl.kernel(
    _passB_body,
    out_type=(
        jax.ShapeDtypeStruct((NC * NHALF, CC), jnp.float32),
        jax.ShapeDtypeStruct((NW * 3 * NPAD,), jnp.float32),
    ),
    mesh=_MESH,
    compiler_params=pltpu.CompilerParams(needs_layout_passes=False),
    scratch_types=[
        pltpu.VMEM((NPAD,), jnp.float32),
        pltpu.VMEM((NPAD,), jnp.float32),
        pltpu.VMEM((3 * NPAD,), jnp.float32),
        pltpu.VMEM((CB,), jnp.int32),
        pltpu.VMEM((CB,), jnp.int32),
        pltpu.VMEM((CB,), jnp.int32),
        pltpu.VMEM((CB, 2), jnp.float32),
        pltpu.VMEM((CB,), jnp.float32),
        pltpu.VMEM((CB, CC), jnp.float32),
        pltpu.VMEM((CB, CC), jnp.float32),
        pltpu.VMEM_SHARED((NACC, CC), jnp.float32),
        pltpu.SemaphoreType.DMA,
    ],
)


def _mm_kernel(act, h_ref, w_ref, b_ref, o_ref):
    y = jnp.dot(h_ref[...], w_ref[...], preferred_element_type=jnp.float32) + b_ref[...]
    if act:
        y = jnp.where(y >= 0, y, 0.01 * y)
    o_ref[...] = y


def _dense(h, W, b, act=False):
    return pl.pallas_call(
        functools.partial(_mm_kernel, act),
        out_shape=jax.ShapeDtypeStruct((h.shape[0], W.shape[1]), jnp.float32),
        grid=(10,),
        in_specs=[
            pl.BlockSpec((h.shape[0] // 10, h.shape[1]), lambda i: (i, 0)),
            pl.BlockSpec((W.shape[0], W.shape[1]), lambda i: (0, 0)),
            pl.BlockSpec((W.shape[1],), lambda i: (0,)),
        ],
        out_specs=pl.BlockSpec((h.shape[0] // 10, W.shape[1]), lambda i: (i, 0)),
    )(h, W, b)


def _qeT_kernel(q_ref, we_ref, o_ref):
    o_ref[...] = lax.dot_general(we_ref[...], q_ref[...],
                                 (((1,), (1,)), ((), ())),
                                 preferred_element_type=jnp.float32)


def _qeT(qs, W_e):
    return pl.pallas_call(
        _qeT_kernel,
        out_shape=jax.ShapeDtypeStruct((2, NN), jnp.float32),
    )(qs, W_e)


def _combine_kernel(accm_ref, meta2_ref, skip_ref, we_ref, o_ref):
    main = accm_ref[:NN]
    meta = meta2_ref[...].reshape(NW, 3 * NPAD)[:NS].sum(axis=0)
    den = meta[:NN]
    s0 = meta[NPAD:NPAD + NN]
    s1 = meta[2 * NPAD:2 * NPAD + NN]
    sw = s0[:, None] * we_ref[0][None, :] + s1[:, None] * we_ref[1][None, :]
    out = (main + sw) / (den[:, None] + 1e-16) + skip_ref[...]
    o_ref[...] = jnp.where(out >= 0, out, 0.01 * out)


def _combine(accm, meta2, skip, W_e):
    return pl.pallas_call(
        _combine_kernel,
        out_shape=jax.ShapeDtypeStruct((NN, CC), jnp.float32),
    )(accm, meta2, skip, W_e)


def kernel(x, edge_index, edge_attr, W_f, b_f, W_q, b_q, W_k, b_k, W_v, b_v,
           W_e, W_skip, b_skip):
    inv = 1.0 / jnp.sqrt(float(CC))
    Wq = W_q * inv
    bq = b_q * inv

    ei_flat = edge_index.reshape(2 * EE)
    h = _dense(x, W_f, b_f, act=True)
    for _ in range(3):
        qs = _dense(h, Wq, bq)
        k = _dense(h, W_k, b_k)
        v = _dense(h, W_v, b_v)
        skip = _dense(h, W_skip, b_skip)
        qeT = _qeT(qs, W_e).reshape(2 * NN)
        alpha, amax2 = _passA(qs, k, qeT, ei_flat, edge_attr)
        accm, meta2 = _passB(v, ei_flat, edge_attr, alpha, amax2)
        h = _combine(accm, meta2, skip, W_e)
    return h


# speedup vs baseline: 7.2227x; 2.2752x over previous
import functools

import jax
import jax.numpy as jnp
from jax import lax
from jax.experimental import pallas as pl
from jax.experimental.pallas import tpu as pltpu
from jax.experimental.pallas import tpu_sc as plsc

NN = 10000
EE = 320000
CC = 128
NPAD = 10240
NC, NS = 2, 16
NW = NC * NS
EPW = EE // NW
CB = 80
NG = CB // 16
NCHUNK = EPW // CB
SLICE = NPAD // NS
ACCW = 144

_MESH = plsc.VectorSubcoreMesh(core_axis_name="c", subcore_axis_name="s",
                               num_cores=NC, num_subcores=NS)

_LANE = lambda: lax.iota(jnp.int32, 16)
_GDN = lax.GatherDimensionNumbers(offset_dims=(), collapsed_slice_dims=(0,),
                                  start_index_map=(0,))


def _take16(x, i):
    return lax.gather(x, i[:, None], _GDN, (1,),
                      mode=lax.GatherScatterMode.PROMISE_IN_BOUNDS)


def _seg_max_update(amax_ref, d16, a16):
    lane = _LANE()
    k16, v16 = plsc.sort_key_val(d16, a16)
    for sh in (1, 2, 4, 8):
        idx = jnp.maximum(lane - sh, 0)
        pk = _take16(k16, idx)
        pv = _take16(v16, idx)
        v16 = jnp.where((pk == k16) & (lane >= sh), jnp.maximum(v16, pv), v16)
    nxt = _take16(k16, jnp.minimum(lane + 1, 15))
    last = (lane == 15) | (k16 != nxt)
    old = plsc.load_gather(amax_ref, [k16])
    plsc.store_scatter(amax_ref, [k16], jnp.maximum(old, v16), mask=last)


SUPA = 2000
NSUPA = EPW // SUPA
NCHA = SUPA // CB
NPAIRA = (NCHA - 1) // 2


def _passA_body(qs_hbm, k_hbm, qeT_hbm, ei_hbm, ea_hbm,
                alpha_hbm, amax2_hbm,
                qe0, qe1, amax, dsta, srca, ea_p, alpha_p, qrA, krA, qrB, krB,
                red, stage, sqA, skA, sqB, skB):
    c = lax.axis_index("c")
    s = lax.axis_index("s")
    wid = c * NS + s
    ebase = wid * EPW
    lane = _LANE()
    zero = jnp.zeros((16,), jnp.float32)
    zi = jnp.zeros((16,), jnp.int32)

    pltpu.sync_copy(qeT_hbm.at[pl.ds(0, NN)], qe0)
    pltpu.sync_copy(qeT_hbm.at[pl.ds(NN, NN)], qe1)

    neg = jnp.full((16,), -3.0e38, jnp.float32)

    def init_body(i, carry):
        amax[pl.ds(i * 16, 16)] = neg
        return carry

    lax.fori_loop(0, NPAD // 16, init_body, 0)

    def issue(ci, qr, kr, sq, sk):
        pltpu.async_copy(qs_hbm.at[dsta.at[pl.ds(ci * CB, CB)]], qr, sq)
        pltpu.async_copy(k_hbm.at[srca.at[pl.ds(ci * CB, CB)]], kr, sk)

    def waitg(qr, kr, sq, sk):
        pltpu.make_async_copy(qs_hbm.at[dsta.at[pl.ds(0, CB)]], qr, sq).wait()
        pltpu.make_async_copy(k_hbm.at[srca.at[pl.ds(0, CB)]], kr, sk).wait()

    def compute(ci, qr, kr, eoff):
        def gb(g, carry2):
            def eb(e, avec):
                ea_ = g * 16 + e
                acc = qr[ea_, pl.ds(0, 16)] * kr[ea_, pl.ds(0, 16)]
                for r in range(1, 8):
                    acc = acc + qr[ea_, pl.ds(r * 16, 16)] * kr[ea_, pl.ds(r * 16, 16)]
                dot = jnp.sum(acc)
                return jnp.where(lane == e, jnp.full((16,), dot, jnp.float32),
                                 avec)

            avec = lax.fori_loop(0, 16, eb, zero)
            gbase = ci * CB + g * 16
            d16 = dsta[pl.ds(gbase, 16)]
            e16 = eoff + g * 16 + lane
            ea0 = plsc.load_gather(ea_p, [e16, zi])
            ea1 = plsc.load_gather(ea_p, [e16, zi + 1])
            q0 = plsc.load_gather(qe0, [d16])
            q1 = plsc.load_gather(qe1, [d16])
            a16 = avec + q0 * ea0 + q1 * ea1
            alpha_p[pl.ds(eoff + g * 16, 16)] = a16
            _seg_max_update(amax, d16, a16)
            return carry2

        lax.fori_loop(0, NG, gb, 0)

    def sup_body(sp, carry):
        base_sp = ebase + sp * SUPA
        pltpu.sync_copy(ei_hbm.at[pl.ds(EE + base_sp, SUPA)], dsta)
        pltpu.sync_copy(ei_hbm.at[pl.ds(base_sp, SUPA)], srca)
        issue(0, qrA, krA, sqA, skA)
        issue(1, qrB, krB, sqB, skB)

        def pair_body(p, carry2):
            cA = 2 * p
            cB = 2 * p + 1
            pltpu.sync_copy(ea_hbm.at[pl.ds(base_sp + p * 2 * CB, 2 * CB), :],
                            ea_p)
            waitg(qrA, krA, sqA, skA)
            compute(cA, qrA, krA, 0)
            issue(jnp.minimum(cA + 2, NCHA - 1), qrA, krA, sqA, skA)
            waitg(qrB, krB, sqB, skB)
            compute(cB, qrB, krB, CB)
            issue(jnp.minimum(cB + 2, NCHA - 1), qrB, krB, sqB, skB)
            pltpu.sync_copy(alpha_p,
                            alpha_hbm.at[pl.ds(base_sp + p * 2 * CB, 2 * CB)])
            return carry2

        lax.fori_loop(0, NPAIRA, pair_body, 0)
        pltpu.sync_copy(ea_hbm.at[pl.ds(base_sp + (NCHA - 1) * CB, CB), :],
                        ea_p.at[pl.ds(0, CB), :])
        waitg(qrA, krA, sqA, skA)
        compute(NCHA - 1, qrA, krA, 0)
        pltpu.sync_copy(alpha_p.at[pl.ds(0, CB)],
                        alpha_hbm.at[pl.ds(base_sp + (NCHA - 1) * CB, CB)])
        waitg(qrB, krB, sqB, skB)
        return carry

    lax.fori_loop(0, NSUPA, sup_body, 0)

    pltpu.sync_copy(amax, stage.at[pl.ds(s * NPAD, NPAD)])
    plsc.subcore_barrier()
    for b in range(4):
        for j in range(4):
            pltpu.sync_copy(stage.at[pl.ds((b * 4 + j) * NPAD + s * SLICE, SLICE)],
                            red.at[pl.ds(j * SLICE, SLICE)])

        def red_body(i, carry, first=(b == 0)):
            m = red[pl.ds(i * 16, 16)]
            for j in range(1, 4):
                m = jnp.maximum(m, red[pl.ds(j * SLICE + i * 16, 16)])
            if not first:
                m = jnp.maximum(m, amax[pl.ds(s * SLICE + i * 16, 16)])
            amax[pl.ds(s * SLICE + i * 16, 16)] = m
            return carry

        lax.fori_loop(0, SLICE // 16, red_body, 0)
    pltpu.sync_copy(amax.at[pl.ds(s * SLICE, SLICE)],
                    amax2_hbm.at[pl.ds(c * NPAD + s * SLICE, SLICE)])


_passA = pl.kernel(
    _passA_body,
    out_type=(
        jax.ShapeDtypeStruct((EE,), jnp.float32),
        jax.ShapeDtypeStruct((NC * NPAD,), jnp.float32),
    ),
    mesh=_MESH,
    compiler_params=pltpu.CompilerParams(needs_layout_passes=False),
    scratch_types=[
        pltpu.VMEM((NN,), jnp.float32),
        pltpu.VMEM((NN,), jnp.float32),
        pltpu.VMEM((NPAD,), jnp.float32),
        pltpu.VMEM((SUPA,), jnp.int32),
        pltpu.VMEM((SUPA,), jnp.int32),
        pltpu.VMEM((2 * CB, 2), jnp.float32),
        pltpu.VMEM((2 * CB,), jnp.float32),
        pltpu.VMEM((CB, CC), jnp.float32),
        pltpu.VMEM((CB, CC), jnp.float32),
        pltpu.VMEM((CB, CC), jnp.float32),
        pltpu.VMEM((CB, CC), jnp.float32),
        pltpu.VMEM((4 * SLICE,), jnp.float32),
        pltpu.VMEM_SHARED((NS * NPAD,), jnp.float32),
        pltpu.SemaphoreType.DMA,
        pltpu.SemaphoreType.DMA,
        pltpu.SemaphoreType.DMA,
        pltpu.SemaphoreType.DMA,
    ],
)


def _seg_sum_update(meta_ref, d16, vals, offs, cbase, nhalf):
    lane = _LANE()
    k16, perm = plsc.sort_key_val(d16, lane)
    vs = [_take16(v, perm) for v in vals]
    for sh in (1, 2, 4, 8):
        idx = jnp.maximum(lane - sh, 0)
        pk = _take16(k16, idx)
        m = (pk == k16) & (lane >= sh)
        vs = [v + jnp.where(m, _take16(v, idx), 0.0) for v in vs]
    nxt = _take16(k16, jnp.minimum(lane + 1, 15))
    inr = (k16 >= cbase) & (k16 < cbase + nhalf)
    last = ((lane == 15) | (k16 != nxt)) & inr
    kidx = jnp.where(inr, k16 - cbase, 0)
    for v, off in zip(vs, offs):
        plsc.addupdate_scatter(meta_ref, [kidx + off], v, mask=last)


EPWB = EE // NS
NHALF = NPAD // 2
SLH = NHALF // NS
NACC = 5184
SUPB = 4000
NSUPB = EPWB // SUPB
NCHB = SUPB // CB
NPAIRB = NCHB // 2


def _passB_body(v_hbm, ei_hbm, ea_hbm, alpha_hbm, amax2_hbm,
                accout_hbm, meta2_hbm,
                amax, tmpa2, meta, dsts, srcs, sidx, ea_p, alpha_p,
                vrA, vrB, buf, exb, acc, svA, svB):
    c = lax.axis_index("c")
    s = lax.axis_index("s")
    wid = c * NS + s
    ebase = s * EPWB
    lane = _LANE()
    zero = jnp.zeros((16,), jnp.float32)
    zi = jnp.zeros((16,), jnp.int32)
    cbase = c * NHALF
    trash = NHALF + s

    pltpu.sync_copy(amax2_hbm.at[pl.ds(0, NPAD)], amax)
    for t in range(4):
        pltpu.sync_copy(amax2_hbm.at[pl.ds(NPAD + t * 2560, 2560)], tmpa2)

        def maxb(i, carry, t=t):
            off = t * 2560 + i * 16
            amax[pl.ds(off, 16)] = jnp.maximum(amax[pl.ds(off, 16)],
                                               tmpa2[pl.ds(i * 16, 16)])
            return carry

        lax.fori_loop(0, 160, maxb, 0)

    def zm(i, carry):
        meta[pl.ds(i * 16, 16)] = zero
        return carry

    lax.fori_loop(0, 3 * NHALF // 16, zm, 0)

    def zb(e, carry):
        for r in range(CC // 16):
            buf[e, pl.ds(r * 16, 16)] = zero
        return carry

    lax.fori_loop(0, CB, zb, 0)
    for j in range(SLH // CB):
        pltpu.sync_copy(buf, acc.at[pl.ds(s * SLH + j * CB, CB), :])
    pl.when(s == 0)(lambda: pltpu.sync_copy(
        buf.at[pl.ds(0, NACC - NHALF), :], acc.at[pl.ds(NHALF, NACC - NHALF), :]))
    plsc.subcore_barrier()

    def issue(ci, vr, sv):
        return pltpu.async_copy(v_hbm.at[srcs.at[pl.ds(ci * CB, CB)]], vr, sv)

    def compute(ci, vr, eoff):
        def gb(g, carry2):
            gbase = ci * CB + g * 16
            d16 = dsts[pl.ds(gbase, 16)]
            m16 = plsc.load_gather(amax, [d16])
            a16 = alpha_p[pl.ds(eoff + g * 16, 16)]
            ex16 = jnp.exp(a16 - m16)
            e16 = eoff + g * 16 + lane
            ea0 = plsc.load_gather(ea_p, [e16, zi])
            ea1 = plsc.load_gather(ea_p, [e16, zi + 1])
            _seg_sum_update(meta, d16, [ex16, ex16 * ea0, ex16 * ea1],
                            [0, NHALF, 2 * NHALF], cbase, NHALF)
            inr = (d16 >= cbase) & (d16 < cbase + NHALF)
            sidx[pl.ds(g * 16, 16)] = jnp.where(inr, d16 - cbase, trash)
            exb[pl.ds(0, 16)] = ex16
            for e in range(16):
                xsv = plsc.load_gather(exb, [zi + e])
                ea_ = g * 16 + e
                for r in range(CC // 16):
                    buf[ea_, pl.ds(r * 16, 16)] = vr[ea_, pl.ds(r * 16, 16)] * xsv
            return carry2

        lax.fori_loop(0, NG, gb, 0)
        pltpu.sync_copy(buf, acc.at[sidx], add=True)

    def sup_body(sp, carry):
        base_sp = ebase + sp * SUPB
        pltpu.sync_copy(ei_hbm.at[pl.ds(EE + base_sp, SUPB)], dsts)
        pltpu.sync_copy(ei_hbm.at[pl.ds(base_sp, SUPB)], srcs)
        issue(0, vrA, svA)
        issue(1, vrB, svB)

        def pair_body(p, carry2):
            cA = 2 * p
            cB = 2 * p + 1
            pltpu.sync_copy(ea_hbm.at[pl.ds(base_sp + p * 2 * CB, 2 * CB), :],
                            ea_p)
            pltpu.sync_copy(alpha_hbm.at[pl.ds(base_sp + p * 2 * CB, 2 * CB)],
                            alpha_p)
            pltpu.make_async_copy(v_hbm.at[srcs.at[pl.ds(0, CB)]], vrA, svA).wait()
            compute(cA, vrA, 0)
            issue(jnp.minimum(cA + 2, NCHB - 1), vrA, svA)
            pltpu.make_async_copy(v_hbm.at[srcs.at[pl.ds(0, CB)]], vrB, svB).wait()
            compute(cB, vrB, CB)
            issue(jnp.minimum(cB + 2, NCHB - 1), vrB, svB)
            return carry2

        lax.fori_loop(0, NPAIRB, pair_body, 0)
        pltpu.make_async_copy(v_hbm.at[srcs.at[pl.ds(0, CB)]], vrA, svA).wait()
        pltpu.make_async_copy(v_hbm.at[srcs.at[pl.ds(0, CB)]], vrB, svB).wait()
        return carry

    lax.fori_loop(0, NSUPB, sup_body, 0)

    pltpu.sync_copy(meta, meta2_hbm.at[pl.ds(wid * 3 * NHALF, 3 * NHALF)])
    plsc.subcore_barrier()
    pltpu.sync_copy(acc.at[pl.ds(s * SLH, SLH), :],
                    accout_hbm.at[pl.ds(c * NHALF + s * SLH, SLH), :])


_passB = pl.kernel(
    _passB_body,
    out_type=(
        jax.ShapeDtypeStruct((NC * NHALF, CC), jnp.float32),
        jax.ShapeDtypeStruct((NW * 3 * NHALF,), jnp.float32),
    ),
    mesh=_MESH,
    compiler_params=pltpu.CompilerParams(needs_layout_passes=False),
    scratch_types=[
        pltpu.VMEM((NPAD,), jnp.float32),
        pltpu.VMEM((2560,), jnp.float32),
        pltpu.VMEM((3 * NHALF,), jnp.float32),
        pltpu.VMEM((SUPB,), jnp.int32),
        pltpu.VMEM((SUPB,), jnp.int32),
        pltpu.VMEM((CB,), jnp.int32),
        pltpu.VMEM((2 * CB, 2), jnp.float32),
        pltpu.VMEM((2 * CB,), jnp.float32),
        pltpu.VMEM((CB, CC), jnp.float32),
        pltpu.VMEM((CB, CC), jnp.float32),
        pltpu.VMEM((CB, CC), jnp.float32),
        pltpu.VMEM((16,), jnp.float32),
        pltpu.VMEM_SHARED((NACC, CC), jnp.float32),
        pltpu.SemaphoreType.DMA,
        pltpu.SemaphoreType.DMA,
    ],
)


def _mm_kernel(act, h_ref, w_ref, b_ref, o_ref):
    y = jnp.dot(h_ref[...], w_ref[...], preferred_element_type=jnp.float32) + b_ref[...]
    if act:
        y = jnp.where(y >= 0, y, 0.01 * y)
    o_ref[...] = y


def _dense(h, W, b, act=False):
    return pl.pallas_call(
        functools.partial(_mm_kernel, act),
        out_shape=jax.ShapeDtypeStruct((h.shape[0], W.shape[1]), jnp.float32),
        grid=(10,),
        in_specs=[
            pl.BlockSpec((h.shape[0] // 10, h.shape[1]), lambda i: (i, 0)),
            pl.BlockSpec((W.shape[0], W.shape[1]), lambda i: (0, 0)),
            pl.BlockSpec((W.shape[1],), lambda i: (0,)),
        ],
        out_specs=pl.BlockSpec((h.shape[0] // 10, W.shape[1]), lambda i: (i, 0)),
    )(h, W, b)


def _qeT_kernel(q_ref, we_ref, o_ref):
    o_ref[...] = lax.dot_general(we_ref[...], q_ref[...],
                                 (((1,), (1,)), ((), ())),
                                 preferred_element_type=jnp.float32)


def _qeT(qs, W_e):
    return pl.pallas_call(
        _qeT_kernel,
        out_shape=jax.ShapeDtypeStruct((2, NN), jnp.float32),
    )(qs, W_e)


def _combine_kernel(accm_ref, meta2_ref, skip_ref, we_ref, o_ref):
    main = accm_ref[:NN]
    m = meta2_ref[...].reshape(NW, 3 * NHALF)
    lo = m[:NS].sum(axis=0)
    hi = m[NS:].sum(axis=0)
    den = jnp.concatenate([lo[:NHALF], hi[:NHALF]])[:NN]
    s0 = jnp.concatenate([lo[NHALF:2 * NHALF], hi[NHALF:2 * NHALF]])[:NN]
    s1 = jnp.concatenate([lo[2 * NHALF:], hi[2 * NHALF:]])[:NN]
    sw = s0[:, None] * we_ref[0][None, :] + s1[:, None] * we_ref[1][None, :]
    out = (main + sw) / (den[:, None] + 1e-16) + skip_ref[...]
    o_ref[...] = jnp.where(out >= 0, out, 0.01 * out)


def _combine(accm, meta2, skip, W_e):
    return pl.pallas_call(
        _combine_kernel,
        out_shape=jax.ShapeDtypeStruct((NN, CC), jnp.float32),
    )(accm, meta2, skip, W_e)


def kernel(x, edge_index, edge_attr, W_f, b_f, W_q, b_q, W_k, b_k, W_v, b_v,
           W_e, W_skip, b_skip):
    inv = 1.0 / jnp.sqrt(float(CC))
    Wq = W_q * inv
    bq = b_q * inv

    ei_flat = edge_index.reshape(2 * EE)
    h = _dense(x, W_f, b_f, act=True)
    for _ in range(3):
        qs = _dense(h, Wq, bq)
        k = _dense(h, W_k, b_k)
        v = _dense(h, W_v, b_v)
        skip = _dense(h, W_skip, b_skip)
        qeT = _qeT(qs, W_e).reshape(2 * NN)
        alpha, amax2 = _passA(qs, k, qeT, ei_flat, edge_attr)
        accm, meta2 = _passB(v, ei_flat, edge_attr, alpha, amax2)
        h = _combine(accm, meta2, skip, W_e)
    return h


# speedup vs baseline: 7.7015x; 1.0663x over previous
import functools

import jax
import jax.numpy as jnp
from jax import lax
from jax.experimental import pallas as pl
from jax.experimental.pallas import tpu as pltpu
from jax.experimental.pallas import tpu_sc as plsc

NN = 10000
EE = 320000
CC = 128
NPAD = 10240
NC, NS = 2, 16
NW = NC * NS
EPW = EE // NW
CB = 80
NG = CB // 16
NCHUNK = EPW // CB
SLICE = NPAD // NS
ACCW = 144

_MESH = plsc.VectorSubcoreMesh(core_axis_name="c", subcore_axis_name="s",
                               num_cores=NC, num_subcores=NS)

_LANE = lambda: lax.iota(jnp.int32, 16)
_GDN = lax.GatherDimensionNumbers(offset_dims=(), collapsed_slice_dims=(0,),
                                  start_index_map=(0,))


def _take16(x, i):
    return lax.gather(x, i[:, None], _GDN, (1,),
                      mode=lax.GatherScatterMode.PROMISE_IN_BOUNDS)


def _seg_max_update(amax_ref, d16, a16):
    lane = _LANE()
    k16, v16 = plsc.sort_key_val(d16, a16)
    for sh in (1, 2, 4, 8):
        idx = jnp.maximum(lane - sh, 0)
        pk = _take16(k16, idx)
        pv = _take16(v16, idx)
        v16 = jnp.where((pk == k16) & (lane >= sh), jnp.maximum(v16, pv), v16)
    nxt = _take16(k16, jnp.minimum(lane + 1, 15))
    last = (lane == 15) | (k16 != nxt)
    old = plsc.load_gather(amax_ref, [k16])
    plsc.store_scatter(amax_ref, [k16], jnp.maximum(old, v16), mask=last)


SUPA = 2000
NSUPA = EPW // SUPA
NCHA = SUPA // CB
NPAIRA = (NCHA - 1) // 2


def _passA_body(qs_hbm, k_hbm, qeT_hbm, ei_hbm, ea_hbm,
                alpha_hbm, amax2_hbm,
                qe0, qe1, amax, dsta, srca, ea_p, alpha_p, qrA, krA, qrB, krB,
                dgiA, sgiA, dgiB, sgiB, red, stage, sqA, skA, sqB, skB):
    c = lax.axis_index("c")
    s = lax.axis_index("s")
    wid = c * NS + s
    ebase = wid * EPW
    lane = _LANE()
    zero = jnp.zeros((16,), jnp.float32)
    zi = jnp.zeros((16,), jnp.int32)

    pltpu.sync_copy(qeT_hbm.at[pl.ds(0, NN)], qe0)
    pltpu.sync_copy(qeT_hbm.at[pl.ds(NN, NN)], qe1)

    neg = jnp.full((16,), -3.0e38, jnp.float32)

    def init_body(i, carry):
        amax[pl.ds(i * 16, 16)] = neg
        return carry

    lax.fori_loop(0, NPAD // 16, init_body, 0)

    def issue(ci, qr, kr, dgi, sgi, sq, sk):
        def cp(i, carry):
            dgi[pl.ds(i * 16, 16)] = dsta[pl.ds(ci * CB + i * 16, 16)]
            sgi[pl.ds(i * 16, 16)] = srca[pl.ds(ci * CB + i * 16, 16)]
            return carry

        lax.fori_loop(0, NG, cp, 0)
        pltpu.async_copy(qs_hbm.at[dgi], qr, sq)
        pltpu.async_copy(k_hbm.at[sgi], kr, sk)

    def waitg(qr, kr, dgi, sgi, sq, sk):
        pltpu.make_async_copy(qs_hbm.at[dgi], qr, sq).wait()
        pltpu.make_async_copy(k_hbm.at[sgi], kr, sk).wait()

    def compute(ci, qr, kr, eoff):
        def gb(g, carry2):
            def eb(e, avec):
                ea_ = g * 16 + e
                acc = qr[ea_, pl.ds(0, 16)] * kr[ea_, pl.ds(0, 16)]
                for r in range(1, 8):
                    acc = acc + qr[ea_, pl.ds(r * 16, 16)] * kr[ea_, pl.ds(r * 16, 16)]
                dot = jnp.sum(acc)
                return jnp.where(lane == e, jnp.full((16,), dot, jnp.float32),
                                 avec)

            avec = lax.fori_loop(0, 16, eb, zero)
            gbase = ci * CB + g * 16
            d16 = dsta[pl.ds(gbase, 16)]
            e16 = eoff + g * 16 + lane
            ea0 = plsc.load_gather(ea_p, [e16, zi])
            ea1 = plsc.load_gather(ea_p, [e16, zi + 1])
            q0 = plsc.load_gather(qe0, [d16])
            q1 = plsc.load_gather(qe1, [d16])
            a16 = avec + q0 * ea0 + q1 * ea1
            alpha_p[pl.ds(eoff + g * 16, 16)] = a16
            _seg_max_update(amax, d16, a16)
            return carry2

        lax.fori_loop(0, NG, gb, 0)

    def sup_body(sp, carry):
        base_sp = ebase + sp * SUPA
        pltpu.sync_copy(ei_hbm.at[pl.ds(EE + base_sp, SUPA)], dsta)
        pltpu.sync_copy(ei_hbm.at[pl.ds(base_sp, SUPA)], srca)
        issue(0, qrA, krA, dgiA, sgiA, sqA, skA)
        issue(1, qrB, krB, dgiB, sgiB, sqB, skB)

        def pair_body(p, carry2):
            cA = 2 * p
            cB = 2 * p + 1
            pltpu.sync_copy(ea_hbm.at[pl.ds(base_sp + p * 2 * CB, 2 * CB), :],
                            ea_p)
            waitg(qrA, krA, dgiA, sgiA, sqA, skA)
            compute(cA, qrA, krA, 0)
            issue(jnp.minimum(cA + 2, NCHA - 1), qrA, krA, dgiA, sgiA, sqA, skA)
            waitg(qrB, krB, dgiB, sgiB, sqB, skB)
            compute(cB, qrB, krB, CB)
            issue(jnp.minimum(cB + 2, NCHA - 1), qrB, krB, dgiB, sgiB, sqB, skB)
            pltpu.sync_copy(alpha_p,
                            alpha_hbm.at[pl.ds(base_sp + p * 2 * CB, 2 * CB)])
            return carry2

        lax.fori_loop(0, NPAIRA, pair_body, 0)
        pltpu.sync_copy(ea_hbm.at[pl.ds(base_sp + (NCHA - 1) * CB, CB), :],
                        ea_p.at[pl.ds(0, CB), :])
        waitg(qrA, krA, dgiA, sgiA, sqA, skA)
        compute(NCHA - 1, qrA, krA, 0)
        pltpu.sync_copy(alpha_p.at[pl.ds(0, CB)],
                        alpha_hbm.at[pl.ds(base_sp + (NCHA - 1) * CB, CB)])
        waitg(qrB, krB, dgiB, sgiB, sqB, skB)
        return carry

    lax.fori_loop(0, NSUPA, sup_body, 0)

    pltpu.sync_copy(amax, stage.at[pl.ds(s * NPAD, NPAD)])
    plsc.subcore_barrier()
    for b in range(4):
        for j in range(4):
            pltpu.sync_copy(stage.at[pl.ds((b * 4 + j) * NPAD + s * SLICE, SLICE)],
                            red.at[pl.ds(j * SLICE, SLICE)])

        def red_body(i, carry, first=(b == 0)):
            m = red[pl.ds(i * 16, 16)]
            for j in range(1, 4):
                m = jnp.maximum(m, red[pl.ds(j * SLICE + i * 16, 16)])
            if not first:
                m = jnp.maximum(m, amax[pl.ds(s * SLICE + i * 16, 16)])
            amax[pl.ds(s * SLICE + i * 16, 16)] = m
            return carry

        lax.fori_loop(0, SLICE // 16, red_body, 0)
    pltpu.sync_copy(amax.at[pl.ds(s * SLICE, SLICE)],
                    amax2_hbm.at[pl.ds(c * NPAD + s * SLICE, SLICE)])


_passA = pl.kernel(
    _passA_body,
    out_type=(
        jax.ShapeDtypeStruct((EE,), jnp.float32),
        jax.ShapeDtypeStruct((NC * NPAD,), jnp.float32),
    ),
    mesh=_MESH,
    compiler_params=pltpu.CompilerParams(needs_layout_passes=False),
    scratch_types=[
        pltpu.VMEM((NN,), jnp.float32),
        pltpu.VMEM((NN,), jnp.float32),
        pltpu.VMEM((NPAD,), jnp.float32),
        pltpu.VMEM((SUPA,), jnp.int32),
        pltpu.VMEM((SUPA,), jnp.int32),
        pltpu.VMEM((2 * CB, 2), jnp.float32),
        pltpu.VMEM((2 * CB,), jnp.float32),
        pltpu.VMEM((CB, CC), jnp.float32),
        pltpu.VMEM((CB, CC), jnp.float32),
        pltpu.VMEM((CB, CC), jnp.float32),
        pltpu.VMEM((CB, CC), jnp.float32),
        pltpu.VMEM((CB,), jnp.int32),
        pltpu.VMEM((CB,), jnp.int32),
        pltpu.VMEM((CB,), jnp.int32),
        pltpu.VMEM((CB,), jnp.int32),
        pltpu.VMEM((4 * SLICE,), jnp.float32),
        pltpu.VMEM_SHARED((NS * NPAD,), jnp.float32),
        pltpu.SemaphoreType.DMA,
        pltpu.SemaphoreType.DMA,
        pltpu.SemaphoreType.DMA,
        pltpu.SemaphoreType.DMA,
    ],
)


def _seg_sum_update(meta_ref, d16, vals, offs, cbase, nhalf):
    lane = _LANE()
    k16, perm = plsc.sort_key_val(d16, lane)
    vs = [_take16(v, perm) for v in vals]
    for sh in (1, 2, 4, 8):
        idx = jnp.maximum(lane - sh, 0)
        pk = _take16(k16, idx)
        m = (pk == k16) & (lane >= sh)
        vs = [v + jnp.where(m, _take16(v, idx), 0.0) for v in vs]
    nxt = _take16(k16, jnp.minimum(lane + 1, 15))
    inr = (k16 >= cbase) & (k16 < cbase + nhalf)
    last = ((lane == 15) | (k16 != nxt)) & inr
    kidx = jnp.where(inr, k16 - cbase, 0)
    for v, off in zip(vs, offs):
        plsc.addupdate_scatter(meta_ref, [kidx + off], v, mask=last)


EPWB = EE // NS
NHALF = NPAD // 2
SLH = NHALF // NS
NACC = 5184
SUPB = 4000
NSUPB = EPWB // SUPB
NCHB = SUPB // CB
NPAIRB = NCHB // 2


def _passB_body(v_hbm, ei_hbm, ea_hbm, alpha_hbm, amax2_hbm,
                accout_hbm, meta2_hbm,
                amax, tmpa2, meta, dsts, srcs, sidx, sgiA, sgiB, ea_p, alpha_p,
                vrA, vrB, buf, exb, acc, svA, svB):
    c = lax.axis_index("c")
    s = lax.axis_index("s")
    wid = c * NS + s
    ebase = s * EPWB
    lane = _LANE()
    zero = jnp.zeros((16,), jnp.float32)
    zi = jnp.zeros((16,), jnp.int32)
    cbase = c * NHALF
    trash = NHALF + s

    pltpu.sync_copy(amax2_hbm.at[pl.ds(0, NPAD)], amax)
    for t in range(4):
        pltpu.sync_copy(amax2_hbm.at[pl.ds(NPAD + t * 2560, 2560)], tmpa2)

        def maxb(i, carry, t=t):
            off = t * 2560 + i * 16
            amax[pl.ds(off, 16)] = jnp.maximum(amax[pl.ds(off, 16)],
                                               tmpa2[pl.ds(i * 16, 16)])
            return carry

        lax.fori_loop(0, 160, maxb, 0)

    def zm(i, carry):
        meta[pl.ds(i * 16, 16)] = zero
        return carry

    lax.fori_loop(0, 3 * NHALF // 16, zm, 0)

    def zb(e, carry):
        for r in range(CC // 16):
            buf[e, pl.ds(r * 16, 16)] = zero
        return carry

    lax.fori_loop(0, CB, zb, 0)
    for j in range(SLH // CB):
        pltpu.sync_copy(buf, acc.at[pl.ds(s * SLH + j * CB, CB), :])
    pl.when(s == 0)(lambda: pltpu.sync_copy(
        buf.at[pl.ds(0, NACC - NHALF), :], acc.at[pl.ds(NHALF, NACC - NHALF), :]))
    plsc.subcore_barrier()

    def issue(ci, vr, sgi, sv):
        def cp(i, carry):
            sgi[pl.ds(i * 16, 16)] = srcs[pl.ds(ci * CB + i * 16, 16)]
            return carry

        lax.fori_loop(0, NG, cp, 0)
        pltpu.async_copy(v_hbm.at[sgi], vr, sv)

    def compute(ci, vr, eoff):
        def gb(g, carry2):
            gbase = ci * CB + g * 16
            d16 = dsts[pl.ds(gbase, 16)]
            m16 = plsc.load_gather(amax, [d16])
            a16 = alpha_p[pl.ds(eoff + g * 16, 16)]
            ex16 = jnp.exp(a16 - m16)
            e16 = eoff + g * 16 + lane
            ea0 = plsc.load_gather(ea_p, [e16, zi])
            ea1 = plsc.load_gather(ea_p, [e16, zi + 1])
            _seg_sum_update(meta, d16, [ex16, ex16 * ea0, ex16 * ea1],
                            [0, NHALF, 2 * NHALF], cbase, NHALF)
            inr = (d16 >= cbase) & (d16 < cbase + NHALF)
            sidx[pl.ds(g * 16, 16)] = jnp.where(inr, d16 - cbase, trash)
            for e in range(16):
                xsv = jnp.sum(jnp.where(lane == e, ex16, zero))
                ea_ = g * 16 + e
                for r in range(CC // 16):
                    buf[ea_, pl.ds(r * 16, 16)] = vr[ea_, pl.ds(r * 16, 16)] * xsv
            return carry2

        lax.fori_loop(0, NG, gb, 0)
        pltpu.sync_copy(buf, acc.at[sidx], add=True)

    def sup_body(sp, carry):
        base_sp = ebase + sp * SUPB
        pltpu.sync_copy(ei_hbm.at[pl.ds(EE + base_sp, SUPB)], dsts)
        pltpu.sync_copy(ei_hbm.at[pl.ds(base_sp, SUPB)], srcs)
        issue(0, vrA, sgiA, svA)
        issue(1, vrB, sgiB, svB)

        def pair_body(p, carry2):
            cA = 2 * p
            cB = 2 * p + 1
            pltpu.sync_copy(ea_hbm.at[pl.ds(base_sp + p * 2 * CB, 2 * CB), :],
                            ea_p)
            pltpu.sync_copy(alpha_hbm.at[pl.ds(base_sp + p * 2 * CB, 2 * CB)],
                            alpha_p)
            pltpu.make_async_copy(v_hbm.at[sgiA], vrA, svA).wait()
            compute(cA, vrA, 0)
            issue(jnp.minimum(cA + 2, NCHB - 1), vrA, sgiA, svA)
            pltpu.make_async_copy(v_hbm.at[sgiB], vrB, svB).wait()
            compute(cB, vrB, CB)
            issue(jnp.minimum(cB + 2, NCHB - 1), vrB, sgiB, svB)
            return carry2

        lax.fori_loop(0, NPAIRB, pair_body, 0)
        pltpu.make_async_copy(v_hbm.at[sgiA], vrA, svA).wait()
        pltpu.make_async_copy(v_hbm.at[sgiB], vrB, svB).wait()
        return carry

    lax.fori_loop(0, NSUPB, sup_body, 0)

    pltpu.sync_copy(meta, meta2_hbm.at[pl.ds(wid * 3 * NHALF, 3 * NHALF)])
    plsc.subcore_barrier()
    pltpu.sync_copy(acc.at[pl.ds(s * SLH, SLH), :],
                    accout_hbm.at[pl.ds(c * NHALF + s * SLH, SLH), :])


_passB = pl.kernel(
    _passB_body,
    out_type=(
        jax.ShapeDtypeStruct((NC * NHALF, CC), jnp.float32),
        jax.ShapeDtypeStruct((NW * 3 * NHALF,), jnp.float32),
    ),
    mesh=_MESH,
    compiler_params=pltpu.CompilerParams(needs_layout_passes=False),
    scratch_types=[
        pltpu.VMEM((NPAD,), jnp.float32),
        pltpu.VMEM((2560,), jnp.float32),
        pltpu.VMEM((3 * NHALF,), jnp.float32),
        pltpu.VMEM((SUPB,), jnp.int32),
        pltpu.VMEM((SUPB,), jnp.int32),
        pltpu.VMEM((CB,), jnp.int32),
        pltpu.VMEM((CB,), jnp.int32),
        pltpu.VMEM((CB,), jnp.int32),
        pltpu.VMEM((2 * CB, 2), jnp.float32),
        pltpu.VMEM((2 * CB,), jnp.float32),
        pltpu.VMEM((CB, CC), jnp.float32),
        pltpu.VMEM((CB, CC), jnp.float32),
        pltpu.VMEM((CB, CC), jnp.float32),
        pltpu.VMEM((16,), jnp.float32),
        pltpu.VMEM_SHARED((NACC, CC), jnp.float32),
        pltpu.SemaphoreType.DMA,
        pltpu.SemaphoreType.DMA,
    ],
)


def _mm_kernel(act, h_ref, w_ref, b_ref, o_ref):
    y = jnp.dot(h_ref[...], w_ref[...], preferred_element_type=jnp.float32) + b_ref[...]
    if act:
        y = jnp.where(y >= 0, y, 0.01 * y)
    o_ref[...] = y


def _dense(h, W, b, act=False):
    return pl.pallas_call(
        functools.partial(_mm_kernel, act),
        out_shape=jax.ShapeDtypeStruct((h.shape[0], W.shape[1]), jnp.float32),
        grid=(10,),
        in_specs=[
            pl.BlockSpec((h.shape[0] // 10, h.shape[1]), lambda i: (i, 0)),
            pl.BlockSpec((W.shape[0], W.shape[1]), lambda i: (0, 0)),
            pl.BlockSpec((W.shape[1],), lambda i: (0,)),
        ],
        out_specs=pl.BlockSpec((h.shape[0] // 10, W.shape[1]), lambda i: (i, 0)),
    )(h, W, b)


def _qeT_kernel(q_ref, we_ref, o_ref):
    o_ref[...] = lax.dot_general(we_ref[...], q_ref[...],
                                 (((1,), (1,)), ((), ())),
                                 preferred_element_type=jnp.float32)


def _qeT(qs, W_e):
    return pl.pallas_call(
        _qeT_kernel,
        out_shape=jax.ShapeDtypeStruct((2, NN), jnp.float32),
    )(qs, W_e)


def _combine_kernel(accm_ref, meta2_ref, skip_ref, we_ref, o_ref):
    main = accm_ref[:NN]
    m = meta2_ref[...].reshape(NW, 3 * NHALF)
    lo = m[:NS].sum(axis=0)
    hi = m[NS:].sum(axis=0)
    den = jnp.concatenate([lo[:NHALF], hi[:NHALF]])[:NN]
    s0 = jnp.concatenate([lo[NHALF:2 * NHALF], hi[NHALF:2 * NHALF]])[:NN]
    s1 = jnp.concatenate([lo[2 * NHALF:], hi[2 * NHALF:]])[:NN]
    sw = s0[:, None] * we_ref[0][None, :] + s1[:, None] * we_ref[1][None, :]
    out = (main + sw) / (den[:, None] + 1e-16) + skip_ref[...]
    o_ref[...] = jnp.where(out >= 0, out, 0.01 * out)


def _combine(accm, meta2, skip, W_e):
    return pl.pallas_call(
        _combine_kernel,
        out_shape=jax.ShapeDtypeStruct((NN, CC), jnp.float32),
    )(accm, meta2, skip, W_e)


def kernel(x, edge_index, edge_attr, W_f, b_f, W_q, b_q, W_k, b_k, W_v, b_v,
           W_e, W_skip, b_skip):
    inv = 1.0 / jnp.sqrt(float(CC))
    Wq = W_q * inv
    bq = b_q * inv

    ei_flat = edge_index.reshape(2 * EE)
    h = _dense(x, W_f, b_f, act=True)
    for _ in range(3):
        qs = _dense(h, Wq, bq)
        k = _dense(h, W_k, b_k)
        v = _dense(h, W_v, b_v)
        skip = _dense(h, W_skip, b_skip)
        qeT = _qeT(qs, W_e).reshape(2 * NN)
        alpha, amax2 = _passA(qs, k, qeT, ei_flat, edge_attr)
        accm, meta2 = _passB(v, ei_flat, edge_attr, alpha, amax2)
        h = _combine(accm, meta2, skip, W_e)
    return h


# speedup vs baseline: 8.0181x; 1.0411x over previous
import functools

import jax
import jax.numpy as jnp
from jax import lax
from jax.experimental import pallas as pl
from jax.experimental.pallas import tpu as pltpu
from jax.experimental.pallas import tpu_sc as plsc

NN = 10000
EE = 320000
CC = 128
NPAD = 10240
NC, NS = 2, 16
NW = NC * NS
EPW = EE // NW
CB = 80
NG = CB // 16
NCHUNK = EPW // CB
SLICE = NPAD // NS
ACCW = 144

_MESH = plsc.VectorSubcoreMesh(core_axis_name="c", subcore_axis_name="s",
                               num_cores=NC, num_subcores=NS)

_LANE = lambda: lax.iota(jnp.int32, 16)
_GDN = lax.GatherDimensionNumbers(offset_dims=(), collapsed_slice_dims=(0,),
                                  start_index_map=(0,))


def _take16(x, i):
    return lax.gather(x, i[:, None], _GDN, (1,),
                      mode=lax.GatherScatterMode.PROMISE_IN_BOUNDS)


def _seg_max_update(amax_ref, d16, a16):
    lane = _LANE()
    k16, v16 = plsc.sort_key_val(d16, a16)
    for sh in (1, 2, 4, 8):
        idx = jnp.maximum(lane - sh, 0)
        pk = _take16(k16, idx)
        pv = _take16(v16, idx)
        v16 = jnp.where((pk == k16) & (lane >= sh), jnp.maximum(v16, pv), v16)
    nxt = _take16(k16, jnp.minimum(lane + 1, 15))
    last = (lane == 15) | (k16 != nxt)
    old = plsc.load_gather(amax_ref, [k16])
    plsc.store_scatter(amax_ref, [k16], jnp.maximum(old, v16), mask=last)


SUPA = 2000
NSUPA = EPW // SUPA
NCHA = SUPA // CB
NPAIRA = (NCHA - 1) // 2


def _passA_body(qs_hbm, k_hbm, qeT_hbm, ei_hbm, ea_hbm,
                alpha_hbm, amax2_hbm,
                qe0, qe1, amax, dsta, srca, ea_p, alpha_p, qrA, krA, qrB, krB,
                dgiA, sgiA, dgiB, sgiB, red, stage, sqA, skA, sqB, skB):
    c = lax.axis_index("c")
    s = lax.axis_index("s")
    wid = c * NS + s
    ebase = wid * EPW
    lane = _LANE()
    zero = jnp.zeros((16,), jnp.float32)
    zi = jnp.zeros((16,), jnp.int32)

    pltpu.sync_copy(qeT_hbm.at[pl.ds(0, NN)], qe0)
    pltpu.sync_copy(qeT_hbm.at[pl.ds(NN, NN)], qe1)

    neg = jnp.full((16,), -3.0e38, jnp.float32)

    def init_body(i, carry):
        amax[pl.ds(i * 16, 16)] = neg
        return carry

    lax.fori_loop(0, NPAD // 16, init_body, 0)

    def issue(ci, qr, kr, dgi, sgi, sq, sk):
        def cp(i, carry):
            dgi[pl.ds(i * 16, 16)] = dsta[pl.ds(ci * CB + i * 16, 16)]
            sgi[pl.ds(i * 16, 16)] = srca[pl.ds(ci * CB + i * 16, 16)]
            return carry

        lax.fori_loop(0, NG, cp, 0)
        pltpu.async_copy(qs_hbm.at[dgi], qr, sq)
        pltpu.async_copy(k_hbm.at[sgi], kr, sk)

    def waitg(qr, kr, dgi, sgi, sq, sk):
        pltpu.make_async_copy(qs_hbm.at[dgi], qr, sq).wait()
        pltpu.make_async_copy(k_hbm.at[sgi], kr, sk).wait()

    def compute(ci, qr, kr, eoff):
        def gb(g, carry2):
            def eb(e, avec):
                ea_ = g * 16 + e
                acc = qr[ea_, pl.ds(0, 16)] * kr[ea_, pl.ds(0, 16)]
                for r in range(1, 8):
                    acc = acc + qr[ea_, pl.ds(r * 16, 16)] * kr[ea_, pl.ds(r * 16, 16)]
                dot = jnp.sum(acc)
                return jnp.where(lane == e, jnp.full((16,), dot, jnp.float32),
                                 avec)

            avec = lax.fori_loop(0, 16, eb, zero)
            gbase = ci * CB + g * 16
            d16 = dsta[pl.ds(gbase, 16)]
            e16 = eoff + g * 16 + lane
            ea0 = plsc.load_gather(ea_p, [e16, zi])
            ea1 = plsc.load_gather(ea_p, [e16, zi + 1])
            q0 = plsc.load_gather(qe0, [d16])
            q1 = plsc.load_gather(qe1, [d16])
            a16 = avec + q0 * ea0 + q1 * ea1
            alpha_p[pl.ds(eoff + g * 16, 16)] = a16
            _seg_max_update(amax, d16, a16)
            return carry2

        lax.fori_loop(0, NG, gb, 0)

    def sup_body(sp, carry):
        base_sp = ebase + sp * SUPA
        pltpu.sync_copy(ei_hbm.at[pl.ds(EE + base_sp, SUPA)], dsta)
        pltpu.sync_copy(ei_hbm.at[pl.ds(base_sp, SUPA)], srca)
        issue(0, qrA, krA, dgiA, sgiA, sqA, skA)
        issue(1, qrB, krB, dgiB, sgiB, sqB, skB)

        def pair_body(p, carry2):
            cA = 2 * p
            cB = 2 * p + 1
            pltpu.sync_copy(ea_hbm.at[pl.ds(base_sp + p * 2 * CB, 2 * CB), :],
                            ea_p)
            waitg(qrA, krA, dgiA, sgiA, sqA, skA)
            compute(cA, qrA, krA, 0)
            issue(jnp.minimum(cA + 2, NCHA - 1), qrA, krA, dgiA, sgiA, sqA, skA)
            waitg(qrB, krB, dgiB, sgiB, sqB, skB)
            compute(cB, qrB, krB, CB)
            issue(jnp.minimum(cB + 2, NCHA - 1), qrB, krB, dgiB, sgiB, sqB, skB)
            pltpu.sync_copy(alpha_p,
                            alpha_hbm.at[pl.ds(base_sp + p * 2 * CB, 2 * CB)])
            return carry2

        lax.fori_loop(0, NPAIRA, pair_body, 0)
        pltpu.sync_copy(ea_hbm.at[pl.ds(base_sp + (NCHA - 1) * CB, CB), :],
                        ea_p.at[pl.ds(0, CB), :])
        waitg(qrA, krA, dgiA, sgiA, sqA, skA)
        compute(NCHA - 1, qrA, krA, 0)
        pltpu.sync_copy(alpha_p.at[pl.ds(0, CB)],
                        alpha_hbm.at[pl.ds(base_sp + (NCHA - 1) * CB, CB)])
        waitg(qrB, krB, dgiB, sgiB, sqB, skB)
        return carry

    lax.fori_loop(0, NSUPA, sup_body, 0)

    pltpu.sync_copy(amax, stage.at[pl.ds(s * NPAD, NPAD)])
    plsc.subcore_barrier()
    for b in range(4):
        for j in range(4):
            pltpu.sync_copy(stage.at[pl.ds((b * 4 + j) * NPAD + s * SLICE, SLICE)],
                            red.at[pl.ds(j * SLICE, SLICE)])

        def red_body(i, carry, first=(b == 0)):
            m = red[pl.ds(i * 16, 16)]
            for j in range(1, 4):
                m = jnp.maximum(m, red[pl.ds(j * SLICE + i * 16, 16)])
            if not first:
                m = jnp.maximum(m, amax[pl.ds(s * SLICE + i * 16, 16)])
            amax[pl.ds(s * SLICE + i * 16, 16)] = m
            return carry

        lax.fori_loop(0, SLICE // 16, red_body, 0)
    pltpu.sync_copy(amax.at[pl.ds(s * SLICE, SLICE)],
                    amax2_hbm.at[pl.ds(c * NPAD + s * SLICE, SLICE)])


_passA = pl.kernel(
    _passA_body,
    out_type=(
        jax.ShapeDtypeStruct((EE,), jnp.float32),
        jax.ShapeDtypeStruct((NC * NPAD,), jnp.float32),
    ),
    mesh=_MESH,
    compiler_params=pltpu.CompilerParams(needs_layout_passes=False),
    scratch_types=[
        pltpu.VMEM((NN,), jnp.float32),
        pltpu.VMEM((NN,), jnp.float32),
        pltpu.VMEM((NPAD,), jnp.float32),
        pltpu.VMEM((SUPA,), jnp.int32),
        pltpu.VMEM((SUPA,), jnp.int32),
        pltpu.VMEM((2 * CB, 2), jnp.float32),
        pltpu.VMEM((2 * CB,), jnp.float32),
        pltpu.VMEM((CB, CC), jnp.float32),
        pltpu.VMEM((CB, CC), jnp.float32),
        pltpu.VMEM((CB, CC), jnp.float32),
        pltpu.VMEM((CB, CC), jnp.float32),
        pltpu.VMEM((CB,), jnp.int32),
        pltpu.VMEM((CB,), jnp.int32),
        pltpu.VMEM((CB,), jnp.int32),
        pltpu.VMEM((CB,), jnp.int32),
        pltpu.VMEM((4 * SLICE,), jnp.float32),
        pltpu.VMEM_SHARED((NS * NPAD,), jnp.float32),
        pltpu.SemaphoreType.DMA,
        pltpu.SemaphoreType.DMA,
        pltpu.SemaphoreType.DMA,
        pltpu.SemaphoreType.DMA,
    ],
)


def _seg_sum_update(meta_ref, d16, vals, offs, cbase, nhalf):
    lane = _LANE()
    k16, perm = plsc.sort_key_val(d16, lane)
    vs = [_take16(v, perm) for v in vals]
    for sh in (1, 2, 4, 8):
        idx = jnp.maximum(lane - sh, 0)
        pk = _take16(k16, idx)
        m = (pk == k16) & (lane >= sh)
        vs = [v + jnp.where(m, _take16(v, idx), 0.0) for v in vs]
    nxt = _take16(k16, jnp.minimum(lane + 1, 15))
    inr = (k16 >= cbase) & (k16 < cbase + nhalf)
    last = ((lane == 15) | (k16 != nxt)) & inr
    kidx = jnp.where(inr, k16 - cbase, 0)
    for v, off in zip(vs, offs):
        plsc.addupdate_scatter(meta_ref, [kidx + off], v, mask=last)


EPWB = EE // NS
NHALF = NPAD // 2
SLH = NHALF // NS
NACC = 5184
SUPB = 4000
NSUPB = EPWB // SUPB
NCHB = SUPB // CB
NPAIRB = NCHB // 2


def _passB_body(v_hbm, ei_hbm, ea_hbm, alpha_hbm, amax2_hbm,
                accout_hbm, meta2_hbm,
                amax, tmpa2, meta, dsts, srcs, sidxA, sidxB, sgiA, sgiB,
                ea_p, alpha_p, vrA, vrB, zbuf, acc, svA, svB, ssA, ssB):
    c = lax.axis_index("c")
    s = lax.axis_index("s")
    wid = c * NS + s
    ebase = s * EPWB
    lane = _LANE()
    zero = jnp.zeros((16,), jnp.float32)
    zi = jnp.zeros((16,), jnp.int32)
    cbase = c * NHALF
    trash = NHALF + s

    pltpu.sync_copy(amax2_hbm.at[pl.ds(0, NPAD)], amax)
    for t in range(4):
        pltpu.sync_copy(amax2_hbm.at[pl.ds(NPAD + t * 2560, 2560)], tmpa2)

        def maxb(i, carry, t=t):
            off = t * 2560 + i * 16
            amax[pl.ds(off, 16)] = jnp.maximum(amax[pl.ds(off, 16)],
                                               tmpa2[pl.ds(i * 16, 16)])
            return carry

        lax.fori_loop(0, 160, maxb, 0)

    def zm(i, carry):
        meta[pl.ds(i * 16, 16)] = zero
        return carry

    lax.fori_loop(0, 3 * NHALF // 16, zm, 0)

    def zb(e, carry):
        for r in range(CC // 16):
            zbuf[e, pl.ds(r * 16, 16)] = zero
        return carry

    lax.fori_loop(0, CB, zb, 0)
    for j in range(SLH // CB):
        pltpu.sync_copy(zbuf, acc.at[pl.ds(s * SLH + j * CB, CB), :])
    pl.when(s == 0)(lambda: pltpu.sync_copy(
        zbuf.at[pl.ds(0, NACC - NHALF), :], acc.at[pl.ds(NHALF, NACC - NHALF), :]))
    plsc.subcore_barrier()

    def issue(ci, vr, sgi, sv):
        def cp(i, carry):
            sgi[pl.ds(i * 16, 16)] = srcs[pl.ds(ci * CB + i * 16, 16)]
            return carry

        lax.fori_loop(0, NG, cp, 0)
        pltpu.async_copy(v_hbm.at[sgi], vr, sv)

    def waitg(vr, sgi, sv):
        pltpu.make_async_copy(v_hbm.at[sgi], vr, sv).wait()

    def waits(vr, sidx, ss):
        pltpu.make_async_copy(vr, acc.at[sidx], ss).wait()

    def compute(ci, vr, sidx, eoff):
        def gb(g, carry2):
            gbase = ci * CB + g * 16
            d16 = dsts[pl.ds(gbase, 16)]
            m16 = plsc.load_gather(amax, [d16])
            a16 = alpha_p[pl.ds(eoff + g * 16, 16)]
            ex16 = jnp.exp(a16 - m16)
            e16 = eoff + g * 16 + lane
            ea0 = plsc.load_gather(ea_p, [e16, zi])
            ea1 = plsc.load_gather(ea_p, [e16, zi + 1])
            _seg_sum_update(meta, d16, [ex16, ex16 * ea0, ex16 * ea1],
                            [0, NHALF, 2 * NHALF], cbase, NHALF)
            inr = (d16 >= cbase) & (d16 < cbase + NHALF)
            sidx[pl.ds(g * 16, 16)] = jnp.where(inr, d16 - cbase, trash)
            for e in range(16):
                xsv = jnp.sum(jnp.where(lane == e, ex16, zero))
                ea_ = g * 16 + e
                for r in range(CC // 16):
                    vr[ea_, pl.ds(r * 16, 16)] = vr[ea_, pl.ds(r * 16, 16)] * xsv
            return carry2

        lax.fori_loop(0, NG, gb, 0)

    def sup_body(sp, carry):
        base_sp = ebase + sp * SUPB
        pltpu.sync_copy(ei_hbm.at[pl.ds(EE + base_sp, SUPB)], dsts)
        pltpu.sync_copy(ei_hbm.at[pl.ds(base_sp, SUPB)], srcs)
        issue(0, vrA, sgiA, svA)
        issue(1, vrB, sgiB, svB)

        def pair_body(p, carry2):
            cA = 2 * p
            cB = 2 * p + 1
            pltpu.sync_copy(ea_hbm.at[pl.ds(base_sp + p * 2 * CB, 2 * CB), :],
                            ea_p)
            pltpu.sync_copy(alpha_hbm.at[pl.ds(base_sp + p * 2 * CB, 2 * CB)],
                            alpha_p)
            waitg(vrA, sgiA, svA)
            compute(cA, vrA, sidxA, 0)
            pltpu.async_copy(vrA, acc.at[sidxA], ssA, add=True)
            waitg(vrB, sgiB, svB)
            compute(cB, vrB, sidxB, CB)
            pltpu.async_copy(vrB, acc.at[sidxB], ssB, add=True)
            waits(vrA, sidxA, ssA)
            issue(jnp.minimum(cA + 2, NCHB - 1), vrA, sgiA, svA)
            waits(vrB, sidxB, ssB)
            issue(jnp.minimum(cB + 2, NCHB - 1), vrB, sgiB, svB)
            return carry2

        lax.fori_loop(0, NPAIRB, pair_body, 0)
        waitg(vrA, sgiA, svA)
        waitg(vrB, sgiB, svB)
        return carry

    lax.fori_loop(0, NSUPB, sup_body, 0)

    pltpu.sync_copy(meta, meta2_hbm.at[pl.ds(wid * 3 * NHALF, 3 * NHALF)])
    plsc.subcore_barrier()
    pltpu.sync_copy(acc.at[pl.ds(s * SLH, SLH), :],
                    accout_hbm.at[pl.ds(c * NHALF + s * SLH, SLH), :])


_passB = pl.kernel(
    _passB_body,
    out_type=(
        jax.ShapeDtypeStruct((NC * NHALF, CC), jnp.float32),
        jax.ShapeDtypeStruct((NW * 3 * NHALF,), jnp.float32),
    ),
    mesh=_MESH,
    compiler_params=pltpu.CompilerParams(needs_layout_passes=False),
    scratch_types=[
        pltpu.VMEM((NPAD,), jnp.float32),
        pltpu.VMEM((2560,), jnp.float32),
        pltpu.VMEM((3 * NHALF,), jnp.float32),
        pltpu.VMEM((SUPB,), jnp.int32),
        pltpu.VMEM((SUPB,), jnp.int32),
        pltpu.VMEM((CB,), jnp.int32),
        pltpu.VMEM((CB,), jnp.int32),
        pltpu.VMEM((CB,), jnp.int32),
        pltpu.VMEM((CB,), jnp.int32),
        pltpu.VMEM((2 * CB, 2), jnp.float32),
        pltpu.VMEM((2 * CB,), jnp.float32),
        pltpu.VMEM((CB, CC), jnp.float32),
        pltpu.VMEM((CB, CC), jnp.float32),
        pltpu.VMEM((CB, CC), jnp.float32),
        pltpu.VMEM_SHARED((NACC, CC), jnp.float32),
        pltpu.SemaphoreType.DMA,
        pltpu.SemaphoreType.DMA,
        pltpu.SemaphoreType.DMA,
        pltpu.SemaphoreType.DMA,
    ],
)


def _mm_kernel(act, h_ref, w_ref, b_ref, o_ref):
    y = jnp.dot(h_ref[...], w_ref[...], preferred_element_type=jnp.float32) + b_ref[...]
    if act:
        y = jnp.where(y >= 0, y, 0.01 * y)
    o_ref[...] = y


def _dense(h, W, b, act=False):
    return pl.pallas_call(
        functools.partial(_mm_kernel, act),
        out_shape=jax.ShapeDtypeStruct((h.shape[0], W.shape[1]), jnp.float32),
        grid=(10,),
        in_specs=[
            pl.BlockSpec((h.shape[0] // 10, h.shape[1]), lambda i: (i, 0)),
            pl.BlockSpec((W.shape[0], W.shape[1]), lambda i: (0, 0)),
            pl.BlockSpec((W.shape[1],), lambda i: (0,)),
        ],
        out_specs=pl.BlockSpec((h.shape[0] // 10, W.shape[1]), lambda i: (i, 0)),
    )(h, W, b)


def _qeT_kernel(q_ref, we_ref, o_ref):
    o_ref[...] = lax.dot_general(we_ref[...], q_ref[...],
                                 (((1,), (1,)), ((), ())),
                                 preferred_element_type=jnp.float32)


def _qeT(qs, W_e):
    return pl.pallas_call(
        _qeT_kernel,
        out_shape=jax.ShapeDtypeStruct((2, NN), jnp.float32),
    )(qs, W_e)


def _combine_kernel(accm_ref, meta2_ref, skip_ref, we_ref, o_ref):
    main = accm_ref[:NN]
    m = meta2_ref[...].reshape(NW, 3 * NHALF)
    lo = m[:NS].sum(axis=0)
    hi = m[NS:].sum(axis=0)
    den = jnp.concatenate([lo[:NHALF], hi[:NHALF]])[:NN]
    s0 = jnp.concatenate([lo[NHALF:2 * NHALF], hi[NHALF:2 * NHALF]])[:NN]
    s1 = jnp.concatenate([lo[2 * NHALF:], hi[2 * NHALF:]])[:NN]
    sw = s0[:, None] * we_ref[0][None, :] + s1[:, None] * we_ref[1][None, :]
    out = (main + sw) / (den[:, None] + 1e-16) + skip_ref[...]
    o_ref[...] = jnp.where(out >= 0, out, 0.01 * out)


def _combine(accm, meta2, skip, W_e):
    return pl.pallas_call(
        _combine_kernel,
        out_shape=jax.ShapeDtypeStruct((NN, CC), jnp.float32),
    )(accm, meta2, skip, W_e)


def kernel(x, edge_index, edge_attr, W_f, b_f, W_q, b_q, W_k, b_k, W_v, b_v,
           W_e, W_skip, b_skip):
    inv = 1.0 / jnp.sqrt(float(CC))
    Wq = W_q * inv
    bq = b_q * inv

    ei_flat = edge_index.reshape(2 * EE)
    h = _dense(x, W_f, b_f, act=True)
    for _ in range(3):
        qs = _dense(h, Wq, bq)
        k = _dense(h, W_k, b_k)
        v = _dense(h, W_v, b_v)
        skip = _dense(h, W_skip, b_skip)
        qeT = _qeT(qs, W_e).reshape(2 * NN)
        alpha, amax2 = _passA(qs, k, qeT, ei_flat, edge_attr)
        accm, meta2 = _passB(v, ei_flat, edge_attr, alpha, amax2)
        h = _combine(accm, meta2, skip, W_e)
    return h


# speedup vs baseline: 8.6867x; 1.0834x over previous
import functools

import jax
import jax.numpy as jnp
from jax import lax
from jax.experimental import pallas as pl
from jax.experimental.pallas import tpu as pltpu
from jax.experimental.pallas import tpu_sc as plsc

NN = 10000
EE = 320000
CC = 128
NPAD = 10240
NC, NS = 2, 16
NW = NC * NS
EPW = EE // NW
CB = 80
NG = CB // 16
NCHUNK = EPW // CB
SLICE = NPAD // NS
ACCW = 144

_MESH = plsc.VectorSubcoreMesh(core_axis_name="c", subcore_axis_name="s",
                               num_cores=NC, num_subcores=NS)

_LANE = lambda: lax.iota(jnp.int32, 16)
_GDN = lax.GatherDimensionNumbers(offset_dims=(), collapsed_slice_dims=(0,),
                                  start_index_map=(0,))


def _take16(x, i):
    return lax.gather(x, i[:, None], _GDN, (1,),
                      mode=lax.GatherScatterMode.PROMISE_IN_BOUNDS)


def _seg_max_update(amax_ref, d16, a16):
    lane = _LANE()
    k16, v16 = plsc.sort_key_val(d16, a16)
    for sh in (1, 2, 4, 8):
        idx = jnp.maximum(lane - sh, 0)
        pk = _take16(k16, idx)
        pv = _take16(v16, idx)
        v16 = jnp.where((pk == k16) & (lane >= sh), jnp.maximum(v16, pv), v16)
    nxt = _take16(k16, jnp.minimum(lane + 1, 15))
    last = (lane == 15) | (k16 != nxt)
    old = plsc.load_gather(amax_ref, [k16])
    plsc.store_scatter(amax_ref, [k16], jnp.maximum(old, v16), mask=last)


NCHA = EPW // CB
NPAIRA = (NCHA - 1) // 2


def _passA_body(qs_hbm, k_hbm, qeT_hbm, ei_hbm, ea_hbm,
                alpha_hbm, amax2_hbm,
                qe0, qe1, amax, dsta, srca, ea_p, alpha_p, qrA, krA, qrB, krB,
                dgiA, sgiA, dgiB, sgiB, red, stage, sqA, skA, sqB, skB):
    c = lax.axis_index("c")
    s = lax.axis_index("s")
    wid = c * NS + s
    ebase = wid * EPW
    lane = _LANE()
    zero = jnp.zeros((16,), jnp.float32)
    zi = jnp.zeros((16,), jnp.int32)

    pltpu.sync_copy(qeT_hbm.at[pl.ds(0, NN)], qe0)
    pltpu.sync_copy(qeT_hbm.at[pl.ds(NN, NN)], qe1)

    neg = jnp.full((16,), -3.0e38, jnp.float32)

    def init_body(i, carry):
        amax[pl.ds(i * 16, 16)] = neg
        return carry

    lax.fori_loop(0, NPAD // 16, init_body, 0)

    def issue(ci, qr, kr, dgi, sgi, sq, sk):
        def cp(i, carry):
            dgi[pl.ds(i * 16, 16)] = dsta[pl.ds(ci * CB + i * 16, 16)]
            sgi[pl.ds(i * 16, 16)] = srca[pl.ds(ci * CB + i * 16, 16)]
            return carry

        lax.fori_loop(0, NG, cp, 0)
        pltpu.async_copy(qs_hbm.at[dgi], qr, sq)
        pltpu.async_copy(k_hbm.at[sgi], kr, sk)

    def waitg(qr, kr, dgi, sgi, sq, sk):
        pltpu.make_async_copy(qs_hbm.at[dgi], qr, sq).wait()
        pltpu.make_async_copy(k_hbm.at[sgi], kr, sk).wait()

    def compute(ci, qr, kr, eoff):
        def gb(g, carry2):
            def eb(e, avec):
                ea_ = g * 16 + e
                acc = qr[ea_, pl.ds(0, 16)] * kr[ea_, pl.ds(0, 16)]
                for r in range(1, 8):
                    acc = acc + qr[ea_, pl.ds(r * 16, 16)] * kr[ea_, pl.ds(r * 16, 16)]
                dot = jnp.sum(acc)
                return jnp.where(lane == e, jnp.full((16,), dot, jnp.float32),
                                 avec)

            avec = lax.fori_loop(0, 16, eb, zero)
            gbase = ci * CB + g * 16
            d16 = dsta[pl.ds(gbase, 16)]
            e16 = eoff + g * 16 + lane
            ea0 = plsc.load_gather(ea_p, [e16, zi])
            ea1 = plsc.load_gather(ea_p, [e16, zi + 1])
            q0 = plsc.load_gather(qe0, [d16])
            q1 = plsc.load_gather(qe1, [d16])
            a16 = avec + q0 * ea0 + q1 * ea1
            alpha_p[pl.ds(eoff + g * 16, 16)] = a16
            _seg_max_update(amax, d16, a16)
            return carry2

        lax.fori_loop(0, NG, gb, 0)

    pltpu.sync_copy(ei_hbm.at[pl.ds(EE + ebase, EPW)], dsta)
    pltpu.sync_copy(ei_hbm.at[pl.ds(ebase, EPW)], srca)
    issue(0, qrA, krA, dgiA, sgiA, sqA, skA)
    issue(1, qrB, krB, dgiB, sgiB, sqB, skB)

    def pair_body(p, carry2):
        cA = 2 * p
        cB = 2 * p + 1
        pltpu.sync_copy(ea_hbm.at[pl.ds(ebase + p * 2 * CB, 2 * CB), :], ea_p)
        waitg(qrA, krA, dgiA, sgiA, sqA, skA)
        compute(cA, qrA, krA, 0)
        issue(jnp.minimum(cA + 2, NCHA - 1), qrA, krA, dgiA, sgiA, sqA, skA)
        waitg(qrB, krB, dgiB, sgiB, sqB, skB)
        compute(cB, qrB, krB, CB)
        issue(jnp.minimum(cB + 2, NCHA - 1), qrB, krB, dgiB, sgiB, sqB, skB)
        pltpu.sync_copy(alpha_p, alpha_hbm.at[pl.ds(ebase + p * 2 * CB, 2 * CB)])
        return carry2

    lax.fori_loop(0, NPAIRA, pair_body, 0)
    pltpu.sync_copy(ea_hbm.at[pl.ds(ebase + (NCHA - 1) * CB, CB), :],
                    ea_p.at[pl.ds(0, CB), :])
    waitg(qrA, krA, dgiA, sgiA, sqA, skA)
    compute(NCHA - 1, qrA, krA, 0)
    pltpu.sync_copy(alpha_p.at[pl.ds(0, CB)],
                    alpha_hbm.at[pl.ds(ebase + (NCHA - 1) * CB, CB)])
    waitg(qrB, krB, dgiB, sgiB, sqB, skB)

    pltpu.sync_copy(amax, stage.at[pl.ds(s * NPAD, NPAD)])
    plsc.subcore_barrier()
    for b in range(4):
        for j in range(4):
            pltpu.sync_copy(stage.at[pl.ds((b * 4 + j) * NPAD + s * SLICE, SLICE)],
                            red.at[pl.ds(j * SLICE, SLICE)])

        def red_body(i, carry, first=(b == 0)):
            m = red[pl.ds(i * 16, 16)]
            for j in range(1, 4):
                m = jnp.maximum(m, red[pl.ds(j * SLICE + i * 16, 16)])
            if not first:
                m = jnp.maximum(m, amax[pl.ds(s * SLICE + i * 16, 16)])
            amax[pl.ds(s * SLICE + i * 16, 16)] = m
            return carry

        lax.fori_loop(0, SLICE // 16, red_body, 0)
    pltpu.sync_copy(amax.at[pl.ds(s * SLICE, SLICE)],
                    amax2_hbm.at[pl.ds(c * NPAD + s * SLICE, SLICE)])


_passA = pl.kernel(
    _passA_body,
    out_type=(
        jax.ShapeDtypeStruct((EE,), jnp.float32),
        jax.ShapeDtypeStruct((NC * NPAD,), jnp.float32),
    ),
    mesh=_MESH,
    compiler_params=pltpu.CompilerParams(needs_layout_passes=False),
    scratch_types=[
        pltpu.VMEM((NN,), jnp.float32),
        pltpu.VMEM((NN,), jnp.float32),
        pltpu.VMEM((NPAD,), jnp.float32),
        pltpu.VMEM((EPW,), jnp.int32),
        pltpu.VMEM((EPW,), jnp.int32),
        pltpu.VMEM((2 * CB, 2), jnp.float32),
        pltpu.VMEM((2 * CB,), jnp.float32),
        pltpu.VMEM((CB, CC), jnp.float32),
        pltpu.VMEM((CB, CC), jnp.float32),
        pltpu.VMEM((CB, CC), jnp.float32),
        pltpu.VMEM((CB, CC), jnp.float32),
        pltpu.VMEM((CB,), jnp.int32),
        pltpu.VMEM((CB,), jnp.int32),
        pltpu.VMEM((CB,), jnp.int32),
        pltpu.VMEM((CB,), jnp.int32),
        pltpu.VMEM((4 * SLICE,), jnp.float32),
        pltpu.VMEM_SHARED((NS * NPAD,), jnp.float32),
        pltpu.SemaphoreType.DMA,
        pltpu.SemaphoreType.DMA,
        pltpu.SemaphoreType.DMA,
        pltpu.SemaphoreType.DMA,
    ],
)


def _seg_sum_update(meta_ref, d16, vals, offs, cbase, nhalf):
    lane = _LANE()
    k16, perm = plsc.sort_key_val(d16, lane)
    vs = [_take16(v, perm) for v in vals]
    for sh in (1, 2, 4, 8):
        idx = jnp.maximum(lane - sh, 0)
        pk = _take16(k16, idx)
        m = (pk == k16) & (lane >= sh)
        vs = [v + jnp.where(m, _take16(v, idx), 0.0) for v in vs]
    nxt = _take16(k16, jnp.minimum(lane + 1, 15))
    inr = (k16 >= cbase) & (k16 < cbase + nhalf)
    last = ((lane == 15) | (k16 != nxt)) & inr
    kidx = jnp.where(inr, k16 - cbase, 0)
    for v, off in zip(vs, offs):
        plsc.addupdate_scatter(meta_ref, [kidx + off], v, mask=last)


EPWB = EE // NS
NHALF = NPAD // 2
SLH = NHALF // NS
NACC = 5184
SUPB = 4000
NSUPB = EPWB // SUPB
NCHB = SUPB // CB
NPAIRB = NCHB // 2


def _passB_body(v_hbm, ei_hbm, ea_hbm, alpha_hbm, amax2_hbm,
                accout_hbm, meta2_hbm,
                amax, tmpa2, meta, dsts, srcs, sidxA, sidxB, sgiA, sgiB,
                ea_p, alpha_s, vrA, vrB, zbuf, acc, svA, svB, ssA, ssB):
    c = lax.axis_index("c")
    s = lax.axis_index("s")
    wid = c * NS + s
    ebase = s * EPWB
    lane = _LANE()
    zero = jnp.zeros((16,), jnp.float32)
    zi = jnp.zeros((16,), jnp.int32)
    cbase = c * NHALF
    trash = NHALF + s

    pltpu.sync_copy(amax2_hbm.at[pl.ds(0, NPAD)], amax)
    for t in range(4):
        pltpu.sync_copy(amax2_hbm.at[pl.ds(NPAD + t * 2560, 2560)], tmpa2)

        def maxb(i, carry, t=t):
            off = t * 2560 + i * 16
            amax[pl.ds(off, 16)] = jnp.maximum(amax[pl.ds(off, 16)],
                                               tmpa2[pl.ds(i * 16, 16)])
            return carry

        lax.fori_loop(0, 160, maxb, 0)

    def zm(i, carry):
        meta[pl.ds(i * 16, 16)] = zero
        return carry

    lax.fori_loop(0, 3 * NHALF // 16, zm, 0)

    def zb(e, carry):
        for r in range(CC // 16):
            zbuf[e, pl.ds(r * 16, 16)] = zero
        return carry

    lax.fori_loop(0, 40, zb, 0)
    for j in range(SLH // 40):
        pltpu.sync_copy(zbuf, acc.at[pl.ds(s * SLH + j * 40, 40), :])
    if NACC - NHALF != 64:
        raise ValueError("trash block must be 64 rows")
    @pl.when(s == 0)
    def _zero_trash():
        pltpu.sync_copy(zbuf, acc.at[pl.ds(NHALF, 40), :])
        pltpu.sync_copy(zbuf.at[pl.ds(0, 24), :], acc.at[pl.ds(NHALF + 40, 24), :])
    plsc.subcore_barrier()

    def issue(ci, vr, sgi, sv):
        def cp(i, carry):
            sgi[pl.ds(i * 16, 16)] = srcs[pl.ds(ci * CB + i * 16, 16)]
            return carry

        lax.fori_loop(0, NG, cp, 0)
        pltpu.async_copy(v_hbm.at[sgi], vr, sv)

    def waitg(vr, sgi, sv):
        pltpu.make_async_copy(v_hbm.at[sgi], vr, sv).wait()

    def waits(vr, sidx, ss):
        pltpu.make_async_copy(vr, acc.at[sidx], ss).wait()

    def compute(ci, vr, sidx, eoff):
        def gb(g, carry2):
            gbase = ci * CB + g * 16
            d16 = dsts[pl.ds(gbase, 16)]
            m16 = plsc.load_gather(amax, [d16])
            a16 = alpha_s[pl.ds(gbase, 16)]
            ex16 = jnp.exp(a16 - m16)
            e16 = eoff + g * 16 + lane
            ea0 = plsc.load_gather(ea_p, [e16, zi])
            ea1 = plsc.load_gather(ea_p, [e16, zi + 1])
            _seg_sum_update(meta, d16, [ex16, ex16 * ea0, ex16 * ea1],
                            [0, NHALF, 2 * NHALF], cbase, NHALF)
            inr = (d16 >= cbase) & (d16 < cbase + NHALF)
            sidx[pl.ds(g * 16, 16)] = jnp.where(inr, d16 - cbase, trash)
            for e in range(16):
                xsv = _take16(ex16, zi + e)
                ea_ = g * 16 + e
                for r in range(CC // 16):
                    vr[ea_, pl.ds(r * 16, 16)] = vr[ea_, pl.ds(r * 16, 16)] * xsv
            return carry2

        lax.fori_loop(0, NG, gb, 0)

    def sup_body(sp, carry):
        base_sp = ebase + sp * SUPB
        pltpu.sync_copy(ei_hbm.at[pl.ds(EE + base_sp, SUPB)], dsts)
        pltpu.sync_copy(ei_hbm.at[pl.ds(base_sp, SUPB)], srcs)
        pltpu.sync_copy(alpha_hbm.at[pl.ds(base_sp, SUPB)], alpha_s)
        issue(0, vrA, sgiA, svA)
        issue(1, vrB, sgiB, svB)

        def pair_body(p, carry2):
            cA = 2 * p
            cB = 2 * p + 1
            pltpu.sync_copy(ea_hbm.at[pl.ds(base_sp + p * 2 * CB, 2 * CB), :],
                            ea_p)
            waitg(vrA, sgiA, svA)
            compute(cA, vrA, sidxA, 0)
            pltpu.async_copy(vrA, acc.at[sidxA], ssA, add=True)
            waitg(vrB, sgiB, svB)
            compute(cB, vrB, sidxB, CB)
            pltpu.async_copy(vrB, acc.at[sidxB], ssB, add=True)
            waits(vrA, sidxA, ssA)
            issue(jnp.minimum(cA + 2, NCHB - 1), vrA, sgiA, svA)
            waits(vrB, sidxB, ssB)
            issue(jnp.minimum(cB + 2, NCHB - 1), vrB, sgiB, svB)
            return carry2

        lax.fori_loop(0, NPAIRB, pair_body, 0)
        waitg(vrA, sgiA, svA)
        waitg(vrB, sgiB, svB)
        return carry

    lax.fori_loop(0, NSUPB, sup_body, 0)

    pltpu.sync_copy(meta, meta2_hbm.at[pl.ds(wid * 3 * NHALF, 3 * NHALF)])
    plsc.subcore_barrier()
    pltpu.sync_copy(acc.at[pl.ds(s * SLH, SLH), :],
                    accout_hbm.at[pl.ds(c * NHALF + s * SLH, SLH), :])


_passB = pl.kernel(
    _passB_body,
    out_type=(
        jax.ShapeDtypeStruct((NC * NHALF, CC), jnp.float32),
        jax.ShapeDtypeStruct((NW * 3 * NHALF,), jnp.float32),
    ),
    mesh=_MESH,
    compiler_params=pltpu.CompilerParams(needs_layout_passes=False),
    scratch_types=[
        pltpu.VMEM((NPAD,), jnp.float32),
        pltpu.VMEM((2560,), jnp.float32),
        pltpu.VMEM((3 * NHALF,), jnp.float32),
        pltpu.VMEM((SUPB,), jnp.int32),
        pltpu.VMEM((SUPB,), jnp.int32),
        pltpu.VMEM((CB,), jnp.int32),
        pltpu.VMEM((CB,), jnp.int32),
        pltpu.VMEM((CB,), jnp.int32),
        pltpu.VMEM((CB,), jnp.int32),
        pltpu.VMEM((2 * CB, 2), jnp.float32),
        pltpu.VMEM((SUPB,), jnp.float32),
        pltpu.VMEM((CB, CC), jnp.float32),
        pltpu.VMEM((CB, CC), jnp.float32),
        pltpu.VMEM((40, CC), jnp.float32),
        pltpu.VMEM_SHARED((NACC, CC), jnp.float32),
        pltpu.SemaphoreType.DMA,
        pltpu.SemaphoreType.DMA,
        pltpu.SemaphoreType.DMA,
        pltpu.SemaphoreType.DMA,
    ],
)


def _mm_kernel(act, h_ref, w_ref, b_ref, o_ref):
    y = jnp.dot(h_ref[...], w_ref[...], preferred_element_type=jnp.float32) + b_ref[...]
    if act:
        y = jnp.where(y >= 0, y, 0.01 * y)
    o_ref[...] = y


def _dense(h, W, b, act=False):
    return pl.pallas_call(
        functools.partial(_mm_kernel, act),
        out_shape=jax.ShapeDtypeStruct((h.shape[0], W.shape[1]), jnp.float32),
        grid=(10,),
        in_specs=[
            pl.BlockSpec((h.shape[0] // 10, h.shape[1]), lambda i: (i, 0)),
            pl.BlockSpec((W.shape[0], W.shape[1]), lambda i: (0, 0)),
            pl.BlockSpec((W.shape[1],), lambda i: (0,)),
        ],
        out_specs=pl.BlockSpec((h.shape[0] // 10, W.shape[1]), lambda i: (i, 0)),
    )(h, W, b)


def _qeT_kernel(q_ref, we_ref, o_ref):
    o_ref[...] = lax.dot_general(we_ref[...], q_ref[...],
                                 (((1,), (1,)), ((), ())),
                                 preferred_element_type=jnp.float32)


def _qeT(qs, W_e):
    return pl.pallas_call(
        _qeT_kernel,
        out_shape=jax.ShapeDtypeStruct((2, NN), jnp.float32),
    )(qs, W_e)


def _combine_kernel(accm_ref, meta2_ref, skip_ref, we_ref, o_ref):
    main = accm_ref[:NN]
    m = meta2_ref[...].reshape(NW, 3 * NHALF)
    lo = m[:NS].sum(axis=0)
    hi = m[NS:].sum(axis=0)
    den = jnp.concatenate([lo[:NHALF], hi[:NHALF]])[:NN]
    s0 = jnp.concatenate([lo[NHALF:2 * NHALF], hi[NHALF:2 * NHALF]])[:NN]
    s1 = jnp.concatenate([lo[2 * NHALF:], hi[2 * NHALF:]])[:NN]
    sw = s0[:, None] * we_ref[0][None, :] + s1[:, None] * we_ref[1][None, :]
    out = (main + sw) / (den[:, None] + 1e-16) + skip_ref[...]
    o_ref[...] = jnp.where(out >= 0, out, 0.01 * out)


def _combine(accm, meta2, skip, W_e):
    return pl.pallas_call(
        _combine_kernel,
        out_shape=jax.ShapeDtypeStruct((NN, CC), jnp.float32),
    )(accm, meta2, skip, W_e)


def kernel(x, edge_index, edge_attr, W_f, b_f, W_q, b_q, W_k, b_k, W_v, b_v,
           W_e, W_skip, b_skip):
    inv = 1.0 / jnp.sqrt(float(CC))
    Wq = W_q * inv
    bq = b_q * inv

    ei_flat = edge_index.reshape(2 * EE)
    h = _dense(x, W_f, b_f, act=True)
    for _ in range(3):
        qs = _dense(h, Wq, bq)
        k = _dense(h, W_k, b_k)
        v = _dense(h, W_v, b_v)
        skip = _dense(h, W_skip, b_skip)
        qeT = _qeT(qs, W_e).reshape(2 * NN)
        alpha, amax2 = _passA(qs, k, qeT, ei_flat, edge_attr)
        accm, meta2 = _passB(v, ei_flat, edge_attr, alpha, amax2)
        h = _combine(accm, meta2, skip, W_e)
    return h


# speedup vs baseline: 12.0247x; 1.3843x over previous
import functools

import jax
import jax.numpy as jnp
from jax import lax
from jax.experimental import pallas as pl
from jax.experimental.pallas import tpu as pltpu
from jax.experimental.pallas import tpu_sc as plsc

NN = 10000
EE = 320000
CC = 128
NPAD = 10240
NC, NS = 2, 16
NW = NC * NS
EPW = EE // NW
CB = 80
NG = CB // 16
NCHUNK = EPW // CB
SLICE = NPAD // NS
ACCW = 144

_MESH = plsc.VectorSubcoreMesh(core_axis_name="c", subcore_axis_name="s",
                               num_cores=NC, num_subcores=NS)

_LANE = lambda: lax.iota(jnp.int32, 16)
_GDN = lax.GatherDimensionNumbers(offset_dims=(), collapsed_slice_dims=(0,),
                                  start_index_map=(0,))


def _take16(x, i):
    return lax.gather(x, i[:, None], _GDN, (1,),
                      mode=lax.GatherScatterMode.PROMISE_IN_BOUNDS)


def _seg_max_update(amax_ref, d16, a16):
    lane = _LANE()
    k16, v16 = plsc.sort_key_val(d16, a16)
    for sh in (1, 2, 4, 8):
        idx = jnp.maximum(lane - sh, 0)
        pk = _take16(k16, idx)
        pv = _take16(v16, idx)
        v16 = jnp.where((pk == k16) & (lane >= sh), jnp.maximum(v16, pv), v16)
    nxt = _take16(k16, jnp.minimum(lane + 1, 15))
    last = (lane == 15) | (k16 != nxt)
    old = plsc.load_gather(amax_ref, [k16])
    plsc.store_scatter(amax_ref, [k16], jnp.maximum(old, v16), mask=last)


NCHA = EPW // CB
NPAIRA = (NCHA - 1) // 2


def _passA_body(qs_hbm, k_hbm, qeT_hbm, ei_hbm, ea_hbm,
                alpha_hbm, amax2_hbm,
                qe0, qe1, amax, dsta, srca, ea_p, alpha_p, qrA, krA, qrB, krB,
                dgiA, sgiA, dgiB, sgiB, red, stage, sqA, skA, sqB, skB):
    c = lax.axis_index("c")
    s = lax.axis_index("s")
    wid = c * NS + s
    ebase = wid * EPW
    lane = _LANE()
    zero = jnp.zeros((16,), jnp.float32)
    zi = jnp.zeros((16,), jnp.int32)

    pltpu.sync_copy(qeT_hbm.at[pl.ds(0, NN)], qe0)
    pltpu.sync_copy(qeT_hbm.at[pl.ds(NN, NN)], qe1)

    neg = jnp.full((16,), -3.0e38, jnp.float32)

    def init_body(i, carry):
        amax[pl.ds(i * 16, 16)] = neg
        return carry

    lax.fori_loop(0, NPAD // 16, init_body, 0)

    def issue(ci, qr, kr, dgi, sgi, sq, sk):
        def cp(i, carry):
            dgi[pl.ds(i * 16, 16)] = dsta[pl.ds(ci * CB + i * 16, 16)]
            sgi[pl.ds(i * 16, 16)] = srca[pl.ds(ci * CB + i * 16, 16)]
            return carry

        lax.fori_loop(0, NG, cp, 0)
        pltpu.async_copy(qs_hbm.at[dgi], qr, sq)
        pltpu.async_copy(k_hbm.at[sgi], kr, sk)

    def waitg(qr, kr, dgi, sgi, sq, sk):
        pltpu.make_async_copy(qs_hbm.at[dgi], qr, sq).wait()
        pltpu.make_async_copy(k_hbm.at[sgi], kr, sk).wait()

    def compute(ci, qr, kr, eoff):
        def gb(g, carry2):
            def eb(e, avec):
                ea_ = g * 16 + e
                acc = qr[ea_, pl.ds(0, 16)] * kr[ea_, pl.ds(0, 16)]
                for r in range(1, 8):
                    acc = acc + qr[ea_, pl.ds(r * 16, 16)] * kr[ea_, pl.ds(r * 16, 16)]
                dot = jnp.sum(acc)
                return jnp.where(lane == e, jnp.full((16,), dot, jnp.float32),
                                 avec)

            avec = lax.fori_loop(0, 16, eb, zero)
            gbase = ci * CB + g * 16
            d16 = dsta[pl.ds(gbase, 16)]
            e2 = 2 * (eoff + g * 16 + lane)
            ea0 = plsc.load_gather(ea_p, [e2])
            ea1 = plsc.load_gather(ea_p, [e2 + 1])
            q0 = plsc.load_gather(qe0, [d16])
            q1 = plsc.load_gather(qe1, [d16])
            a16 = avec + q0 * ea0 + q1 * ea1
            alpha_p[pl.ds(eoff + g * 16, 16)] = a16
            _seg_max_update(amax, d16, a16)
            return carry2

        lax.fori_loop(0, NG, gb, 0)

    pltpu.sync_copy(ei_hbm.at[pl.ds(EE + ebase, EPW)], dsta)
    pltpu.sync_copy(ei_hbm.at[pl.ds(ebase, EPW)], srca)
    issue(0, qrA, krA, dgiA, sgiA, sqA, skA)
    issue(1, qrB, krB, dgiB, sgiB, sqB, skB)

    def pair_body(p, carry2):
        cA = 2 * p
        cB = 2 * p + 1
        pltpu.sync_copy(ea_hbm.at[pl.ds(2 * (ebase + p * 2 * CB), 4 * CB)], ea_p)
        waitg(qrA, krA, dgiA, sgiA, sqA, skA)
        compute(cA, qrA, krA, 0)
        issue(jnp.minimum(cA + 2, NCHA - 1), qrA, krA, dgiA, sgiA, sqA, skA)
        waitg(qrB, krB, dgiB, sgiB, sqB, skB)
        compute(cB, qrB, krB, CB)
        issue(jnp.minimum(cB + 2, NCHA - 1), qrB, krB, dgiB, sgiB, sqB, skB)
        pltpu.sync_copy(alpha_p, alpha_hbm.at[pl.ds(ebase + p * 2 * CB, 2 * CB)])
        return carry2

    lax.fori_loop(0, NPAIRA, pair_body, 0)
    pltpu.sync_copy(ea_hbm.at[pl.ds(2 * (ebase + (NCHA - 1) * CB), 2 * CB)],
                    ea_p.at[pl.ds(0, 2 * CB)])
    waitg(qrA, krA, dgiA, sgiA, sqA, skA)
    compute(NCHA - 1, qrA, krA, 0)
    pltpu.sync_copy(alpha_p.at[pl.ds(0, CB)],
                    alpha_hbm.at[pl.ds(ebase + (NCHA - 1) * CB, CB)])
    waitg(qrB, krB, dgiB, sgiB, sqB, skB)

    pltpu.sync_copy(amax, stage.at[pl.ds(s * NPAD, NPAD)])
    plsc.subcore_barrier()
    for b in range(4):
        for j in range(4):
            pltpu.sync_copy(stage.at[pl.ds((b * 4 + j) * NPAD + s * SLICE, SLICE)],
                            red.at[pl.ds(j * SLICE, SLICE)])

        def red_body(i, carry, first=(b == 0)):
            m = red[pl.ds(i * 16, 16)]
            for j in range(1, 4):
                m = jnp.maximum(m, red[pl.ds(j * SLICE + i * 16, 16)])
            if not first:
                m = jnp.maximum(m, amax[pl.ds(s * SLICE + i * 16, 16)])
            amax[pl.ds(s * SLICE + i * 16, 16)] = m
            return carry

        lax.fori_loop(0, SLICE // 16, red_body, 0)
    pltpu.sync_copy(amax.at[pl.ds(s * SLICE, SLICE)],
                    amax2_hbm.at[pl.ds(c * NPAD + s * SLICE, SLICE)])


_passA = pl.kernel(
    _passA_body,
    out_type=(
        jax.ShapeDtypeStruct((EE,), jnp.float32),
        jax.ShapeDtypeStruct((NC * NPAD,), jnp.float32),
    ),
    mesh=_MESH,
    compiler_params=pltpu.CompilerParams(needs_layout_passes=False),
    scratch_types=[
        pltpu.VMEM((NN,), jnp.float32),
        pltpu.VMEM((NN,), jnp.float32),
        pltpu.VMEM((NPAD,), jnp.float32),
        pltpu.VMEM((EPW,), jnp.int32),
        pltpu.VMEM((EPW,), jnp.int32),
        pltpu.VMEM((4 * CB,), jnp.float32),
        pltpu.VMEM((2 * CB,), jnp.float32),
        pltpu.VMEM((CB, CC), jnp.float32),
        pltpu.VMEM((CB, CC), jnp.float32),
        pltpu.VMEM((CB, CC), jnp.float32),
        pltpu.VMEM((CB, CC), jnp.float32),
        pltpu.VMEM((CB,), jnp.int32),
        pltpu.VMEM((CB,), jnp.int32),
        pltpu.VMEM((CB,), jnp.int32),
        pltpu.VMEM((CB,), jnp.int32),
        pltpu.VMEM((4 * SLICE,), jnp.float32),
        pltpu.VMEM_SHARED((NS * NPAD,), jnp.float32),
        pltpu.SemaphoreType.DMA,
        pltpu.SemaphoreType.DMA,
        pltpu.SemaphoreType.DMA,
        pltpu.SemaphoreType.DMA,
    ],
)


def _seg_sum_update(meta_ref, d16, vals, offs, cbase, nhalf):
    lane = _LANE()
    k16, perm = plsc.sort_key_val(d16, lane)
    vs = [_take16(v, perm) for v in vals]
    for sh in (1, 2, 4, 8):
        idx = jnp.maximum(lane - sh, 0)
        pk = _take16(k16, idx)
        m = (pk == k16) & (lane >= sh)
        vs = [v + jnp.where(m, _take16(v, idx), 0.0) for v in vs]
    nxt = _take16(k16, jnp.minimum(lane + 1, 15))
    inr = (k16 >= cbase) & (k16 < cbase + nhalf)
    last = ((lane == 15) | (k16 != nxt)) & inr
    kidx = jnp.where(inr, k16 - cbase, 0)
    for v, off in zip(vs, offs):
        plsc.addupdate_scatter(meta_ref, [kidx + off], v, mask=last)


EPWB = EE // NS
NHALF = NPAD // 2
SLH = NHALF // NS
NACC = 5184
SUPB = 4000
NSUPB = EPWB // SUPB
NCHB = SUPB // CB
NPAIRB = NCHB // 2


def _passB_body(v_hbm, ei_hbm, ea_hbm, alpha_hbm, amax2_hbm,
                accout_hbm, meta2_hbm,
                amax, tmpa2, meta, dsts, srcs, sidxA, sidxB, sgiA, sgiB,
                ea_s, alpha_s, vrA, vrB, acc, svA, svB, ssA, ssB):
    c = lax.axis_index("c")
    s = lax.axis_index("s")
    wid = c * NS + s
    ebase = s * EPWB
    lane = _LANE()
    zero = jnp.zeros((16,), jnp.float32)
    zi = jnp.zeros((16,), jnp.int32)
    cbase = c * NHALF
    trash = NHALF + s

    pltpu.sync_copy(amax2_hbm.at[pl.ds(0, NPAD)], amax)
    for t in range(5):
        pltpu.sync_copy(amax2_hbm.at[pl.ds(NPAD + t * 2048, 2048)], tmpa2)

        def maxb(i, carry, t=t):
            off = t * 2048 + i * 16
            amax[pl.ds(off, 16)] = jnp.maximum(amax[pl.ds(off, 16)],
                                               tmpa2[pl.ds(i * 16, 16)])
            return carry

        lax.fori_loop(0, 128, maxb, 0)

    def zm(i, carry):
        meta[pl.ds(i * 16, 16)] = zero
        return carry

    lax.fori_loop(0, 3 * NHALF // 16, zm, 0)

    def zb(e, carry):
        for r in range(CC // 16):
            vrA[e, pl.ds(r * 16, 16)] = zero
        return carry

    lax.fori_loop(0, CB, zb, 0)
    for j in range(SLH // CB):
        pltpu.sync_copy(vrA, acc.at[pl.ds(s * SLH + j * CB, CB), :])
    if NACC - NHALF != 64:
        raise ValueError("trash block must be 64 rows")
    @pl.when(s == 0)
    def _zero_trash():
        pltpu.sync_copy(vrA.at[pl.ds(0, 64), :], acc.at[pl.ds(NHALF, 64), :])
    plsc.subcore_barrier()

    def issue(ci, vr, sgi, sv):
        def cp(i, carry):
            sgi[pl.ds(i * 16, 16)] = srcs[pl.ds(ci * CB + i * 16, 16)]
            return carry

        lax.fori_loop(0, NG, cp, 0)
        pltpu.async_copy(v_hbm.at[sgi], vr, sv)

    def waitg(vr, sgi, sv):
        pltpu.make_async_copy(v_hbm.at[sgi], vr, sv).wait()

    def waits(vr, sidx, ss):
        pltpu.make_async_copy(vr, acc.at[sidx], ss).wait()

    def compute(ci, vr, sidx):
        def gb(g, carry2):
            gbase = ci * CB + g * 16
            d16 = dsts[pl.ds(gbase, 16)]
            m16 = plsc.load_gather(amax, [d16])
            a16 = alpha_s[pl.ds(gbase, 16)]
            ex16 = jnp.exp(a16 - m16)
            e2 = 2 * (gbase + lane)
            ea0 = plsc.load_gather(ea_s, [e2])
            ea1 = plsc.load_gather(ea_s, [e2 + 1])
            _seg_sum_update(meta, d16, [ex16, ex16 * ea0, ex16 * ea1],
                            [0, NHALF, 2 * NHALF], cbase, NHALF)
            inr = (d16 >= cbase) & (d16 < cbase + NHALF)
            sidx[pl.ds(g * 16, 16)] = jnp.where(inr, d16 - cbase, trash)
            for e in range(16):
                xsv = _take16(ex16, zi + e)
                ea_ = g * 16 + e
                for r in range(CC // 16):
                    vr[ea_, pl.ds(r * 16, 16)] = vr[ea_, pl.ds(r * 16, 16)] * xsv
            return carry2

        lax.fori_loop(0, NG, gb, 0)

    def sup_body(sp, carry):
        base_sp = ebase + sp * SUPB
        pltpu.sync_copy(ei_hbm.at[pl.ds(EE + base_sp, SUPB)], dsts)
        pltpu.sync_copy(ei_hbm.at[pl.ds(base_sp, SUPB)], srcs)
        pltpu.sync_copy(alpha_hbm.at[pl.ds(base_sp, SUPB)], alpha_s)
        pltpu.sync_copy(ea_hbm.at[pl.ds(2 * base_sp, 2 * SUPB)], ea_s)
        issue(0, vrA, sgiA, svA)
        issue(1, vrB, sgiB, svB)

        def pair_body(p, carry2):
            cA = 2 * p
            cB = 2 * p + 1
            waitg(vrA, sgiA, svA)
            compute(cA, vrA, sidxA)
            pltpu.async_copy(vrA, acc.at[sidxA], ssA, add=True)
            waitg(vrB, sgiB, svB)
            compute(cB, vrB, sidxB)
            pltpu.async_copy(vrB, acc.at[sidxB], ssB, add=True)
            waits(vrA, sidxA, ssA)
            issue(jnp.minimum(cA + 2, NCHB - 1), vrA, sgiA, svA)
            waits(vrB, sidxB, ssB)
            issue(jnp.minimum(cB + 2, NCHB - 1), vrB, sgiB, svB)
            return carry2

        lax.fori_loop(0, NPAIRB, pair_body, 0)
        waitg(vrA, sgiA, svA)
        waitg(vrB, sgiB, svB)
        return carry

    lax.fori_loop(0, NSUPB, sup_body, 0)

    pltpu.sync_copy(meta, meta2_hbm.at[pl.ds(wid * 3 * NHALF, 3 * NHALF)])
    plsc.subcore_barrier()
    pltpu.sync_copy(acc.at[pl.ds(s * SLH, SLH), :],
                    accout_hbm.at[pl.ds(c * NHALF + s * SLH, SLH), :])


_passB = pl.kernel(
    _passB_body,
    out_type=(
        jax.ShapeDtypeStruct((NC * NHALF, CC), jnp.float32),
        jax.ShapeDtypeStruct((NW * 3 * NHALF,), jnp.float32),
    ),
    mesh=_MESH,
    compiler_params=pltpu.CompilerParams(needs_layout_passes=False),
    scratch_types=[
        pltpu.VMEM((NPAD,), jnp.float32),
        pltpu.VMEM((2048,), jnp.float32),
        pltpu.VMEM((3 * NHALF,), jnp.float32),
        pltpu.VMEM((SUPB,), jnp.int32),
        pltpu.VMEM((SUPB,), jnp.int32),
        pltpu.VMEM((CB,), jnp.int32),
        pltpu.VMEM((CB,), jnp.int32),
        pltpu.VMEM((CB,), jnp.int32),
        pltpu.VMEM((CB,), jnp.int32),
        pltpu.VMEM((2 * SUPB,), jnp.float32),
        pltpu.VMEM((SUPB,), jnp.float32),
        pltpu.VMEM((CB, CC), jnp.float32),
        pltpu.VMEM((CB, CC), jnp.float32),
        pltpu.VMEM_SHARED((NACC, CC), jnp.float32),
        pltpu.SemaphoreType.DMA,
        pltpu.SemaphoreType.DMA,
        pltpu.SemaphoreType.DMA,
        pltpu.SemaphoreType.DMA,
    ],
)


def _mm_kernel(act, h_ref, w_ref, b_ref, o_ref):
    y = jnp.dot(h_ref[...], w_ref[...], preferred_element_type=jnp.float32) + b_ref[...]
    if act:
        y = jnp.where(y >= 0, y, 0.01 * y)
    o_ref[...] = y


def _dense(h, W, b, act=False):
    return pl.pallas_call(
        functools.partial(_mm_kernel, act),
        out_shape=jax.ShapeDtypeStruct((h.shape[0], W.shape[1]), jnp.float32),
        grid=(10,),
        in_specs=[
            pl.BlockSpec((h.shape[0] // 10, h.shape[1]), lambda i: (i, 0)),
            pl.BlockSpec((W.shape[0], W.shape[1]), lambda i: (0, 0)),
            pl.BlockSpec((W.shape[1],), lambda i: (0,)),
        ],
        out_specs=pl.BlockSpec((h.shape[0] // 10, W.shape[1]), lambda i: (i, 0)),
    )(h, W, b)


def _qeT_kernel(q_ref, we_ref, o_ref):
    o_ref[...] = lax.dot_general(we_ref[...], q_ref[...],
                                 (((1,), (1,)), ((), ())),
                                 preferred_element_type=jnp.float32)


def _qeT(qs, W_e):
    return pl.pallas_call(
        _qeT_kernel,
        out_shape=jax.ShapeDtypeStruct((2, NN), jnp.float32),
    )(qs, W_e)


def _combine_kernel(accm_ref, meta2_ref, skip_ref, we_ref, o_ref):
    main = accm_ref[:NN]
    m = meta2_ref[...].reshape(NW, 3 * NHALF)
    lo = m[:NS].sum(axis=0)
    hi = m[NS:].sum(axis=0)
    den = jnp.concatenate([lo[:NHALF], hi[:NHALF]])[:NN]
    s0 = jnp.concatenate([lo[NHALF:2 * NHALF], hi[NHALF:2 * NHALF]])[:NN]
    s1 = jnp.concatenate([lo[2 * NHALF:], hi[2 * NHALF:]])[:NN]
    sw = s0[:, None] * we_ref[0][None, :] + s1[:, None] * we_ref[1][None, :]
    out = (main + sw) / (den[:, None] + 1e-16) + skip_ref[...]
    o_ref[...] = jnp.where(out >= 0, out, 0.01 * out)


def _combine(accm, meta2, skip, W_e):
    return pl.pallas_call(
        _combine_kernel,
        out_shape=jax.ShapeDtypeStruct((NN, CC), jnp.float32),
    )(accm, meta2, skip, W_e)


def kernel(x, edge_index, edge_attr, W_f, b_f, W_q, b_q, W_k, b_k, W_v, b_v,
           W_e, W_skip, b_skip):
    inv = 1.0 / jnp.sqrt(float(CC))
    Wq = W_q * inv
    bq = b_q * inv

    ei_flat = edge_index.reshape(2 * EE)
    ea_flat = edge_attr.reshape(2 * EE)
    h = _dense(x, W_f, b_f, act=True)
    for _ in range(3):
        qs = _dense(h, Wq, bq)
        k = _dense(h, W_k, b_k)
        v = _dense(h, W_v, b_v)
        skip = _dense(h, W_skip, b_skip)
        qeT = _qeT(qs, W_e).reshape(2 * NN)
        alpha, amax2 = _passA(qs, k, qeT, ei_flat, ea_flat)
        accm, meta2 = _passB(v, ei_flat, ea_flat, alpha, amax2)
        h = _combine(accm, meta2, skip, W_e)
    return h


# speedup vs baseline: 12.1678x; 1.0119x over previous
import functools

import jax
import jax.numpy as jnp
from jax import lax
from jax.experimental import pallas as pl
from jax.experimental.pallas import tpu as pltpu
from jax.experimental.pallas import tpu_sc as plsc

NN = 10000
EE = 320000
CC = 128
NPAD = 10240
NC, NS = 2, 16
NW = NC * NS
EPW = EE // NW
CB = 80
NG = CB // 16
NCHUNK = EPW // CB
SLICE = NPAD // NS
ACCW = 144

_MESH = plsc.VectorSubcoreMesh(core_axis_name="c", subcore_axis_name="s",
                               num_cores=NC, num_subcores=NS)

_LANE = lambda: lax.iota(jnp.int32, 16)
_GDN = lax.GatherDimensionNumbers(offset_dims=(), collapsed_slice_dims=(0,),
                                  start_index_map=(0,))


def _take16(x, i):
    return lax.gather(x, i[:, None], _GDN, (1,),
                      mode=lax.GatherScatterMode.PROMISE_IN_BOUNDS)


def _seg_max_update(amax_ref, d16, a16):
    lane = _LANE()
    k16, v16 = plsc.sort_key_val(d16, a16)
    for sh in (1, 2, 4, 8):
        idx = jnp.maximum(lane - sh, 0)
        pk = _take16(k16, idx)
        pv = _take16(v16, idx)
        v16 = jnp.where((pk == k16) & (lane >= sh), jnp.maximum(v16, pv), v16)
    nxt = _take16(k16, jnp.minimum(lane + 1, 15))
    last = (lane == 15) | (k16 != nxt)
    old = plsc.load_gather(amax_ref, [k16])
    plsc.store_scatter(amax_ref, [k16], jnp.maximum(old, v16), mask=last)


NCHA = EPW // CB
NPAIRA = (NCHA - 1) // 2


def _passA_body(qs_hbm, k_hbm, qeT_hbm, ei_hbm, ea_hbm,
                alpha_hbm, amax2_hbm,
                qe0, qe1, amax, dsta, srca, ea_p, alpha_p, qrA, krA, qrB, krB,
                dgiA, sgiA, dgiB, sgiB, red, stage, sqA, skA, sqB, skB):
    c = lax.axis_index("c")
    s = lax.axis_index("s")
    wid = c * NS + s
    ebase = wid * EPW
    lane = _LANE()
    zero = jnp.zeros((16,), jnp.float32)
    zi = jnp.zeros((16,), jnp.int32)

    pltpu.sync_copy(qeT_hbm.at[pl.ds(0, NN)], qe0)
    pltpu.sync_copy(qeT_hbm.at[pl.ds(NN, NN)], qe1)

    neg = jnp.full((16,), -3.0e38, jnp.float32)

    def init_body(i, carry):
        amax[pl.ds(i * 16, 16)] = neg
        return carry

    lax.fori_loop(0, NPAD // 16, init_body, 0)

    def issue(ci, qr, kr, dgi, sgi, sq, sk):
        def cp(i, carry):
            dgi[pl.ds(i * 16, 16)] = dsta[pl.ds(ci * CB + i * 16, 16)]
            sgi[pl.ds(i * 16, 16)] = srca[pl.ds(ci * CB + i * 16, 16)]
            return carry

        lax.fori_loop(0, NG, cp, 0)
        pltpu.async_copy(qs_hbm.at[dgi], qr, sq)
        pltpu.async_copy(k_hbm.at[sgi], kr, sk)

    def waitg(qr, kr, dgi, sgi, sq, sk):
        pltpu.make_async_copy(qs_hbm.at[dgi], qr, sq).wait()
        pltpu.make_async_copy(k_hbm.at[sgi], kr, sk).wait()

    def compute(ci, qr, kr, eoff):
        def gb(g, carry2):
            def eb(e, avec):
                ea_ = g * 16 + e
                acc = qr[ea_, pl.ds(0, 16)] * kr[ea_, pl.ds(0, 16)]
                for r in range(1, 8):
                    acc = acc + qr[ea_, pl.ds(r * 16, 16)] * kr[ea_, pl.ds(r * 16, 16)]
                dot = jnp.sum(acc)
                return jnp.where(lane == e, jnp.full((16,), dot, jnp.float32),
                                 avec)

            avec = lax.fori_loop(0, 16, eb, zero)
            gbase = ci * CB + g * 16
            d16 = dsta[pl.ds(gbase, 16)]
            e2 = 2 * (eoff + g * 16 + lane)
            ea0 = plsc.load_gather(ea_p, [e2])
            ea1 = plsc.load_gather(ea_p, [e2 + 1])
            q0 = plsc.load_gather(qe0, [d16])
            q1 = plsc.load_gather(qe1, [d16])
            a16 = avec + q0 * ea0 + q1 * ea1
            alpha_p[pl.ds(eoff + g * 16, 16)] = a16
            _seg_max_update(amax, d16, a16)
            return carry2

        lax.fori_loop(0, NG, gb, 0)

    pltpu.sync_copy(ei_hbm.at[pl.ds(EE + ebase, EPW)], dsta)
    pltpu.sync_copy(ei_hbm.at[pl.ds(ebase, EPW)], srca)
    issue(0, qrA, krA, dgiA, sgiA, sqA, skA)
    issue(1, qrB, krB, dgiB, sgiB, sqB, skB)

    def pair_body(p, carry2):
        cA = 2 * p
        cB = 2 * p + 1
        pltpu.sync_copy(ea_hbm.at[pl.ds(2 * (ebase + p * 2 * CB), 4 * CB)], ea_p)
        waitg(qrA, krA, dgiA, sgiA, sqA, skA)
        compute(cA, qrA, krA, 0)
        issue(jnp.minimum(cA + 2, NCHA - 1), qrA, krA, dgiA, sgiA, sqA, skA)
        waitg(qrB, krB, dgiB, sgiB, sqB, skB)
        compute(cB, qrB, krB, CB)
        issue(jnp.minimum(cB + 2, NCHA - 1), qrB, krB, dgiB, sgiB, sqB, skB)
        pltpu.sync_copy(alpha_p, alpha_hbm.at[pl.ds(ebase + p * 2 * CB, 2 * CB)])
        return carry2

    lax.fori_loop(0, NPAIRA, pair_body, 0)
    pltpu.sync_copy(ea_hbm.at[pl.ds(2 * (ebase + (NCHA - 1) * CB), 2 * CB)],
                    ea_p.at[pl.ds(0, 2 * CB)])
    waitg(qrA, krA, dgiA, sgiA, sqA, skA)
    compute(NCHA - 1, qrA, krA, 0)
    pltpu.sync_copy(alpha_p.at[pl.ds(0, CB)],
                    alpha_hbm.at[pl.ds(ebase + (NCHA - 1) * CB, CB)])
    waitg(qrB, krB, dgiB, sgiB, sqB, skB)

    pltpu.sync_copy(amax, stage.at[pl.ds(s * NPAD, NPAD)])
    plsc.subcore_barrier()
    for b in range(4):
        for j in range(4):
            pltpu.sync_copy(stage.at[pl.ds((b * 4 + j) * NPAD + s * SLICE, SLICE)],
                            red.at[pl.ds(j * SLICE, SLICE)])

        def red_body(i, carry, first=(b == 0)):
            m = red[pl.ds(i * 16, 16)]
            for j in range(1, 4):
                m = jnp.maximum(m, red[pl.ds(j * SLICE + i * 16, 16)])
            if not first:
                m = jnp.maximum(m, amax[pl.ds(s * SLICE + i * 16, 16)])
            amax[pl.ds(s * SLICE + i * 16, 16)] = m
            return carry

        lax.fori_loop(0, SLICE // 16, red_body, 0)
    pltpu.sync_copy(amax.at[pl.ds(s * SLICE, SLICE)],
                    amax2_hbm.at[pl.ds(c * NPAD + s * SLICE, SLICE)])


_passA = pl.kernel(
    _passA_body,
    out_type=(
        jax.ShapeDtypeStruct((EE,), jnp.float32),
        jax.ShapeDtypeStruct((NC * NPAD,), jnp.float32),
    ),
    mesh=_MESH,
    compiler_params=pltpu.CompilerParams(needs_layout_passes=False),
    scratch_types=[
        pltpu.VMEM((NN,), jnp.float32),
        pltpu.VMEM((NN,), jnp.float32),
        pltpu.VMEM((NPAD,), jnp.float32),
        pltpu.VMEM((EPW,), jnp.int32),
        pltpu.VMEM((EPW,), jnp.int32),
        pltpu.VMEM((4 * CB,), jnp.float32),
        pltpu.VMEM((2 * CB,), jnp.float32),
        pltpu.VMEM((CB, CC), jnp.float32),
        pltpu.VMEM((CB, CC), jnp.float32),
        pltpu.VMEM((CB, CC), jnp.float32),
        pltpu.VMEM((CB, CC), jnp.float32),
        pltpu.VMEM((CB,), jnp.int32),
        pltpu.VMEM((CB,), jnp.int32),
        pltpu.VMEM((CB,), jnp.int32),
        pltpu.VMEM((CB,), jnp.int32),
        pltpu.VMEM((4 * SLICE,), jnp.float32),
        pltpu.VMEM_SHARED((NS * NPAD,), jnp.float32),
        pltpu.SemaphoreType.DMA,
        pltpu.SemaphoreType.DMA,
        pltpu.SemaphoreType.DMA,
        pltpu.SemaphoreType.DMA,
    ],
)


def _seg_sum_update(meta_ref, d16, vals, offs, cbase, nhalf):
    lane = _LANE()
    k16, perm = plsc.sort_key_val(d16, lane)
    vs = [_take16(v, perm) for v in vals]
    for sh in (1, 2, 4, 8):
        idx = jnp.maximum(lane - sh, 0)
        pk = _take16(k16, idx)
        m = (pk == k16) & (lane >= sh)
        vs = [v + jnp.where(m, _take16(v, idx), 0.0) for v in vs]
    nxt = _take16(k16, jnp.minimum(lane + 1, 15))
    inr = (k16 >= cbase) & (k16 < cbase + nhalf)
    last = ((lane == 15) | (k16 != nxt)) & inr
    kidx = jnp.where(inr, k16 - cbase, 0)
    for v, off in zip(vs, offs):
        plsc.addupdate_scatter(meta_ref, [kidx + off], v, mask=last)


EPWB = EE // NS
NHALF = NPAD // 2
SLH = NHALF // NS
NACC = 5184
SUPB = 4000
NSUPB = EPWB // SUPB
NCHB = SUPB // CB
NPAIRB = NCHB // 2


def _passB_body(v_hbm, ei_hbm, ea_hbm, alpha_hbm, amax2_hbm,
                accout_hbm, meta2_hbm,
                amax, tmpa2, meta, dsts, srcs, sidxA, sidxB, sgiA, sgiB,
                ea_s, alpha_s, vrA, vrB, acc, svA, svB, ssA, ssB):
    c = lax.axis_index("c")
    s = lax.axis_index("s")
    wid = c * NS + s
    ebase = s * EPWB
    lane = _LANE()
    zero = jnp.zeros((16,), jnp.float32)
    zi = jnp.zeros((16,), jnp.int32)
    cbase = c * NHALF
    trash = NHALF + s

    pltpu.sync_copy(amax2_hbm.at[pl.ds(0, NPAD)], amax)
    for t in range(5):
        pltpu.sync_copy(amax2_hbm.at[pl.ds(NPAD + t * 2048, 2048)], tmpa2)

        def maxb(i, carry, t=t):
            off = t * 2048 + i * 16
            amax[pl.ds(off, 16)] = jnp.maximum(amax[pl.ds(off, 16)],
                                               tmpa2[pl.ds(i * 16, 16)])
            return carry

        lax.fori_loop(0, 128, maxb, 0)

    def zm(i, carry):
        meta[pl.ds(i * 16, 16)] = zero
        return carry

    lax.fori_loop(0, 3 * NHALF // 16, zm, 0)

    def zb(e, carry):
        for r in range(CC // 16):
            vrA[e, pl.ds(r * 16, 16)] = zero
        return carry

    lax.fori_loop(0, CB, zb, 0)
    for j in range(SLH // CB):
        pltpu.sync_copy(vrA, acc.at[pl.ds(s * SLH + j * CB, CB), :])
    if NACC - NHALF != 64:
        raise ValueError("trash block must be 64 rows")
    @pl.when(s == 0)
    def _zero_trash():
        pltpu.sync_copy(vrA.at[pl.ds(0, 64), :], acc.at[pl.ds(NHALF, 64), :])
    plsc.subcore_barrier()

    def issue(ci, vr, sgi, sv):
        def cp(i, carry):
            sgi[pl.ds(i * 16, 16)] = srcs[pl.ds(ci * CB + i * 16, 16)]
            return carry

        lax.fori_loop(0, NG, cp, 0)
        pltpu.async_copy(v_hbm.at[sgi], vr, sv)

    def waitg(vr, sgi, sv):
        pltpu.make_async_copy(v_hbm.at[sgi], vr, sv).wait()

    def waits(vr, sidx, ss):
        pltpu.make_async_copy(vr, acc.at[sidx], ss).wait()

    def compute(ci, vr, sidx):
        def gb(g, carry2):
            gbase = ci * CB + g * 16
            d16 = dsts[pl.ds(gbase, 16)]
            m16 = plsc.load_gather(amax, [d16])
            a16 = alpha_s[pl.ds(gbase, 16)]
            ex16 = jnp.exp(a16 - m16)
            e2 = 2 * (gbase + lane)
            ea0 = plsc.load_gather(ea_s, [e2])
            ea1 = plsc.load_gather(ea_s, [e2 + 1])
            _seg_sum_update(meta, d16, [ex16, ex16 * ea0, ex16 * ea1],
                            [0, NHALF, 2 * NHALF], cbase, NHALF)
            inr = (d16 >= cbase) & (d16 < cbase + NHALF)
            sidx[pl.ds(g * 16, 16)] = jnp.where(inr, d16 - cbase, trash)
            for e in range(16):
                xsv = _take16(ex16, zi + e)
                ea_ = g * 16 + e
                for r in range(CC // 16):
                    vr[ea_, pl.ds(r * 16, 16)] = vr[ea_, pl.ds(r * 16, 16)] * xsv
            return carry2

        lax.fori_loop(0, NG, gb, 0)

    def sup_body(sp, carry):
        base_sp = ebase + sp * SUPB
        pltpu.sync_copy(ei_hbm.at[pl.ds(EE + base_sp, SUPB)], dsts)
        pltpu.sync_copy(ei_hbm.at[pl.ds(base_sp, SUPB)], srcs)
        pltpu.sync_copy(alpha_hbm.at[pl.ds(base_sp, SUPB)], alpha_s)
        pltpu.sync_copy(ea_hbm.at[pl.ds(2 * base_sp, 2 * SUPB)], ea_s)
        issue(0, vrA, sgiA, svA)
        issue(1, vrB, sgiB, svB)

        def pair_body(p, carry2):
            cA = 2 * p
            cB = 2 * p + 1
            waitg(vrA, sgiA, svA)
            compute(cA, vrA, sidxA)
            pltpu.async_copy(vrA, acc.at[sidxA], ssA, add=True)
            waitg(vrB, sgiB, svB)
            compute(cB, vrB, sidxB)
            pltpu.async_copy(vrB, acc.at[sidxB], ssB, add=True)
            waits(vrA, sidxA, ssA)
            issue(jnp.minimum(cA + 2, NCHB - 1), vrA, sgiA, svA)
            waits(vrB, sidxB, ssB)
            issue(jnp.minimum(cB + 2, NCHB - 1), vrB, sgiB, svB)
            return carry2

        lax.fori_loop(0, NPAIRB, pair_body, 0)
        waitg(vrA, sgiA, svA)
        waitg(vrB, sgiB, svB)
        return carry

    lax.fori_loop(0, NSUPB, sup_body, 0)

    pltpu.sync_copy(meta, meta2_hbm.at[pl.ds(wid * 3 * NHALF, 3 * NHALF)])
    plsc.subcore_barrier()
    pltpu.sync_copy(acc.at[pl.ds(s * SLH, SLH), :],
                    accout_hbm.at[pl.ds(c * NHALF + s * SLH, SLH), :])


_passB = pl.kernel(
    _passB_body,
    out_type=(
        jax.ShapeDtypeStruct((NC * NHALF, CC), jnp.float32),
        jax.ShapeDtypeStruct((NW * 3 * NHALF,), jnp.float32),
    ),
    mesh=_MESH,
    compiler_params=pltpu.CompilerParams(needs_layout_passes=False),
    scratch_types=[
        pltpu.VMEM((NPAD,), jnp.float32),
        pltpu.VMEM((2048,), jnp.float32),
        pltpu.VMEM((3 * NHALF,), jnp.float32),
        pltpu.VMEM((SUPB,), jnp.int32),
        pltpu.VMEM((SUPB,), jnp.int32),
        pltpu.VMEM((CB,), jnp.int32),
        pltpu.VMEM((CB,), jnp.int32),
        pltpu.VMEM((CB,), jnp.int32),
        pltpu.VMEM((CB,), jnp.int32),
        pltpu.VMEM((2 * SUPB,), jnp.float32),
        pltpu.VMEM((SUPB,), jnp.float32),
        pltpu.VMEM((CB, CC), jnp.float32),
        pltpu.VMEM((CB, CC), jnp.float32),
        pltpu.VMEM_SHARED((NACC, CC), jnp.float32),
        pltpu.SemaphoreType.DMA,
        pltpu.SemaphoreType.DMA,
        pltpu.SemaphoreType.DMA,
        pltpu.SemaphoreType.DMA,
    ],
)


def _mm_kernel(act, h_ref, w_ref, b_ref, o_ref):
    y = jnp.dot(h_ref[...], w_ref[...], preferred_element_type=jnp.float32) + b_ref[...]
    if act:
        y = jnp.where(y >= 0, y, 0.01 * y)
    o_ref[...] = y


def _dense(h, W, b, act=False):
    return pl.pallas_call(
        functools.partial(_mm_kernel, act),
        out_shape=jax.ShapeDtypeStruct((h.shape[0], W.shape[1]), jnp.float32),
        grid=(10,),
        in_specs=[
            pl.BlockSpec((h.shape[0] // 10, h.shape[1]), lambda i: (i, 0)),
            pl.BlockSpec((W.shape[0], W.shape[1]), lambda i: (0, 0)),
            pl.BlockSpec((W.shape[1],), lambda i: (0,)),
        ],
        out_specs=pl.BlockSpec((h.shape[0] // 10, W.shape[1]), lambda i: (i, 0)),
    )(h, W, b)


def _dense4_kernel(h_ref, wq_ref, bq_ref, wk_ref, bk_ref, wv_ref, bv_ref,
                   ws_ref, bs_ref, oq_ref, ok_ref, ov_ref, os_ref):
    h = h_ref[...]
    oq_ref[...] = jnp.dot(h, wq_ref[...], preferred_element_type=jnp.float32) + bq_ref[...]
    ok_ref[...] = jnp.dot(h, wk_ref[...], preferred_element_type=jnp.float32) + bk_ref[...]
    ov_ref[...] = jnp.dot(h, wv_ref[...], preferred_element_type=jnp.float32) + bv_ref[...]
    os_ref[...] = jnp.dot(h, ws_ref[...], preferred_element_type=jnp.float32) + bs_ref[...]


def _dense4(h, Wq, bq, Wk, bk, Wv, bv, Ws, bs):
    wspec = pl.BlockSpec((CC, CC), lambda i: (0, 0))
    bspec = pl.BlockSpec((CC,), lambda i: (0,))
    blk = pl.BlockSpec((NN // 10, CC), lambda i: (i, 0))
    return pl.pallas_call(
        _dense4_kernel,
        out_shape=tuple(jax.ShapeDtypeStruct((NN, CC), jnp.float32)
                        for _ in range(4)),
        grid=(10,),
        in_specs=[blk, wspec, bspec, wspec, bspec, wspec, bspec, wspec, bspec],
        out_specs=(blk, blk, blk, blk),
    )(h, Wq, bq, Wk, bk, Wv, bv, Ws, bs)


def _qeT_kernel(q_ref, we_ref, o_ref):
    o_ref[...] = lax.dot_general(we_ref[...], q_ref[...],
                                 (((1,), (1,)), ((), ())),
                                 preferred_element_type=jnp.float32)


def _qeT(qs, W_e):
    return pl.pallas_call(
        _qeT_kernel,
        out_shape=jax.ShapeDtypeStruct((2, NN), jnp.float32),
    )(qs, W_e)


def _combine_kernel(accm_ref, meta2_ref, skip_ref, we_ref, o_ref):
    main = accm_ref[:NN]
    m = meta2_ref[...].reshape(NW, 3 * NHALF)
    lo = m[:NS].sum(axis=0)
    hi = m[NS:].sum(axis=0)
    den = jnp.concatenate([lo[:NHALF], hi[:NHALF]])[:NN]
    s0 = jnp.concatenate([lo[NHALF:2 * NHALF], hi[NHALF:2 * NHALF]])[:NN]
    s1 = jnp.concatenate([lo[2 * NHALF:], hi[2 * NHALF:]])[:NN]
    sw = s0[:, None] * we_ref[0][None, :] + s1[:, None] * we_ref[1][None, :]
    out = (main + sw) / (den[:, None] + 1e-16) + skip_ref[...]
    o_ref[...] = jnp.where(out >= 0, out, 0.01 * out)


def _combine(accm, meta2, skip, W_e):
    return pl.pallas_call(
        _combine_kernel,
        out_shape=jax.ShapeDtypeStruct((NN, CC), jnp.float32),
    )(accm, meta2, skip, W_e)


def kernel(x, edge_index, edge_attr, W_f, b_f, W_q, b_q, W_k, b_k, W_v, b_v,
           W_e, W_skip, b_skip):
    inv = 1.0 / jnp.sqrt(float(CC))
    Wq = W_q * inv
    bq = b_q * inv

    ei_flat = edge_index.reshape(2 * EE)
    ea_flat = edge_attr.reshape(2 * EE)
    h = _dense(x, W_f, b_f, act=True)
    for _ in range(3):
        qs, k, v, skip = _dense4(h, Wq, bq, W_k, b_k, W_v, b_v, W_skip, b_skip)
        qeT = _qeT(qs, W_e).reshape(2 * NN)
        alpha, amax2 = _passA(qs, k, qeT, ei_flat, ea_flat)
        accm, meta2 = _passB(v, ei_flat, ea_flat, alpha, amax2)
        h = _combine(accm, meta2, skip, W_e)
    return h


# speedup vs baseline: 13.3531x; 1.0974x over previous
import functools

import jax
import jax.numpy as jnp
from jax import lax
from jax.experimental import pallas as pl
from jax.experimental.pallas import tpu as pltpu
from jax.experimental.pallas import tpu_sc as plsc

NN = 10000
EE = 320000
CC = 128
NPAD = 10240
NC, NS = 2, 16
NW = NC * NS
EPW = EE // NW
CB = 80
NG = CB // 16
NCHUNK = EPW // CB
SLICE = NPAD // NS
ACCW = 144

_MESH = plsc.VectorSubcoreMesh(core_axis_name="c", subcore_axis_name="s",
                               num_cores=NC, num_subcores=NS)

_LANE = lambda: lax.iota(jnp.int32, 16)
_GDN = lax.GatherDimensionNumbers(offset_dims=(), collapsed_slice_dims=(0,),
                                  start_index_map=(0,))


def _take16(x, i):
    return lax.gather(x, i[:, None], _GDN, (1,),
                      mode=lax.GatherScatterMode.PROMISE_IN_BOUNDS)


def _seg_max_update(amax_ref, d16, a16):
    lane = _LANE()
    k16, v16 = plsc.sort_key_val(d16, a16)
    for sh in (1, 2, 4, 8):
        idx = jnp.maximum(lane - sh, 0)
        pk = _take16(k16, idx)
        pv = _take16(v16, idx)
        v16 = jnp.where((pk == k16) & (lane >= sh), jnp.maximum(v16, pv), v16)
    nxt = _take16(k16, jnp.minimum(lane + 1, 15))
    last = (lane == 15) | (k16 != nxt)
    old = plsc.load_gather(amax_ref, [k16])
    plsc.store_scatter(amax_ref, [k16], jnp.maximum(old, v16), mask=last)


NCHA = EPW // CB
NPAIRA = (NCHA - 1) // 2


def _passA_body(qs_hbm, k_hbm, qeT_hbm, ei_hbm, ea_hbm,
                alpha_hbm, amax2_hbm,
                qe0, qe1, amax, dsta, srca, ea_p, alpha_p, qrA, krA, qrB, krB,
                dgiA, sgiA, dgiB, sgiB, red, stage, sqA, skA, sqB, skB):
    c = lax.axis_index("c")
    s = lax.axis_index("s")
    wid = c * NS + s
    ebase = wid * EPW
    lane = _LANE()
    zero = jnp.zeros((16,), jnp.float32)
    zi = jnp.zeros((16,), jnp.int32)

    pltpu.sync_copy(qeT_hbm.at[pl.ds(0, NN)], qe0)
    pltpu.sync_copy(qeT_hbm.at[pl.ds(NN, NN)], qe1)

    neg = jnp.full((16,), -3.0e38, jnp.float32)

    def init_body(i, carry):
        amax[pl.ds(i * 16, 16)] = neg
        return carry

    lax.fori_loop(0, NPAD // 16, init_body, 0)

    def issue(ci, qr, kr, dgi, sgi, sq, sk):
        def cp(i, carry):
            dgi[pl.ds(i * 16, 16)] = dsta[pl.ds(ci * CB + i * 16, 16)]
            sgi[pl.ds(i * 16, 16)] = srca[pl.ds(ci * CB + i * 16, 16)]
            return carry

        lax.fori_loop(0, NG, cp, 0)
        pltpu.async_copy(qs_hbm.at[dgi], qr, sq)
        pltpu.async_copy(k_hbm.at[sgi], kr, sk)

    def waitg(qr, kr, dgi, sgi, sq, sk):
        pltpu.make_async_copy(qs_hbm.at[dgi], qr, sq).wait()
        pltpu.make_async_copy(k_hbm.at[sgi], kr, sk).wait()

    def compute(ci, qr, kr, eoff):
        def gb(g, carry2):
            def eb(e, avec):
                ea_ = g * 16 + e
                acc = qr[ea_, pl.ds(0, 16)] * kr[ea_, pl.ds(0, 16)]
                for r in range(1, 8):
                    acc = acc + qr[ea_, pl.ds(r * 16, 16)] * kr[ea_, pl.ds(r * 16, 16)]
                dot = jnp.sum(acc)
                return jnp.where(lane == e, jnp.full((16,), dot, jnp.float32),
                                 avec)

            avec = lax.fori_loop(0, 16, eb, zero)
            gbase = ci * CB + g * 16
            d16 = dsta[pl.ds(gbase, 16)]
            e2 = 2 * (eoff + g * 16 + lane)
            ea0 = plsc.load_gather(ea_p, [e2])
            ea1 = plsc.load_gather(ea_p, [e2 + 1])
            q0 = plsc.load_gather(qe0, [d16])
            q1 = plsc.load_gather(qe1, [d16])
            a16 = avec + q0 * ea0 + q1 * ea1
            alpha_p[pl.ds(eoff + g * 16, 16)] = a16
            _seg_max_update(amax, d16, a16)
            return carry2

        lax.fori_loop(0, NG, gb, 0)

    pltpu.sync_copy(ei_hbm.at[pl.ds(EE + ebase, EPW)], dsta)
    pltpu.sync_copy(ei_hbm.at[pl.ds(ebase, EPW)], srca)
    issue(0, qrA, krA, dgiA, sgiA, sqA, skA)
    issue(1, qrB, krB, dgiB, sgiB, sqB, skB)

    def pair_body(p, carry2):
        cA = 2 * p
        cB = 2 * p + 1
        pltpu.sync_copy(ea_hbm.at[pl.ds(2 * (ebase + p * 2 * CB), 4 * CB)], ea_p)
        waitg(qrA, krA, dgiA, sgiA, sqA, skA)
        compute(cA, qrA, krA, 0)
        issue(jnp.minimum(cA + 2, NCHA - 1), qrA, krA, dgiA, sgiA, sqA, skA)
        waitg(qrB, krB, dgiB, sgiB, sqB, skB)
        compute(cB, qrB, krB, CB)
        issue(jnp.minimum(cB + 2, NCHA - 1), qrB, krB, dgiB, sgiB, sqB, skB)
        pltpu.sync_copy(alpha_p, alpha_hbm.at[pl.ds(ebase + p * 2 * CB, 2 * CB)])
        return carry2

    lax.fori_loop(0, NPAIRA, pair_body, 0)
    pltpu.sync_copy(ea_hbm.at[pl.ds(2 * (ebase + (NCHA - 1) * CB), 2 * CB)],
                    ea_p.at[pl.ds(0, 2 * CB)])
    waitg(qrA, krA, dgiA, sgiA, sqA, skA)
    compute(NCHA - 1, qrA, krA, 0)
    pltpu.sync_copy(alpha_p.at[pl.ds(0, CB)],
                    alpha_hbm.at[pl.ds(ebase + (NCHA - 1) * CB, CB)])
    waitg(qrB, krB, dgiB, sgiB, sqB, skB)

    pltpu.sync_copy(amax, stage.at[pl.ds(s * NPAD, NPAD)])
    plsc.subcore_barrier()
    for b in range(4):
        for j in range(4):
            pltpu.sync_copy(stage.at[pl.ds((b * 4 + j) * NPAD + s * SLICE, SLICE)],
                            red.at[pl.ds(j * SLICE, SLICE)])

        def red_body(i, carry, first=(b == 0)):
            m = red[pl.ds(i * 16, 16)]
            for j in range(1, 4):
                m = jnp.maximum(m, red[pl.ds(j * SLICE + i * 16, 16)])
            if not first:
                m = jnp.maximum(m, amax[pl.ds(s * SLICE + i * 16, 16)])
            amax[pl.ds(s * SLICE + i * 16, 16)] = m
            return carry

        lax.fori_loop(0, SLICE // 16, red_body, 0)
    pltpu.sync_copy(amax.at[pl.ds(s * SLICE, SLICE)],
                    amax2_hbm.at[pl.ds(c * NPAD + s * SLICE, SLICE)])


_passA = pl.kernel(
    _passA_body,
    out_type=(
        jax.ShapeDtypeStruct((EE,), jnp.float32),
        jax.ShapeDtypeStruct((NC * NPAD,), jnp.float32),
    ),
    mesh=_MESH,
    compiler_params=pltpu.CompilerParams(needs_layout_passes=False),
    scratch_types=[
        pltpu.VMEM((NN,), jnp.float32),
        pltpu.VMEM((NN,), jnp.float32),
        pltpu.VMEM((NPAD,), jnp.float32),
        pltpu.VMEM((EPW,), jnp.int32),
        pltpu.VMEM((EPW,), jnp.int32),
        pltpu.VMEM((4 * CB,), jnp.float32),
        pltpu.VMEM((2 * CB,), jnp.float32),
        pltpu.VMEM((CB, CC), jnp.float32),
        pltpu.VMEM((CB, CC), jnp.float32),
        pltpu.VMEM((CB, CC), jnp.float32),
        pltpu.VMEM((CB, CC), jnp.float32),
        pltpu.VMEM((CB,), jnp.int32),
        pltpu.VMEM((CB,), jnp.int32),
        pltpu.VMEM((CB,), jnp.int32),
        pltpu.VMEM((CB,), jnp.int32),
        pltpu.VMEM((4 * SLICE,), jnp.float32),
        pltpu.VMEM_SHARED((NS * NPAD,), jnp.float32),
        pltpu.SemaphoreType.DMA,
        pltpu.SemaphoreType.DMA,
        pltpu.SemaphoreType.DMA,
        pltpu.SemaphoreType.DMA,
    ],
)


def _seg_sum_update(meta_ref, d16, vals, offs, cbase, nhalf):
    lane = _LANE()
    k16, perm = plsc.sort_key_val(d16, lane)
    vs = [_take16(v, perm) for v in vals]
    for sh in (1, 2, 4, 8):
        idx = jnp.maximum(lane - sh, 0)
        pk = _take16(k16, idx)
        m = (pk == k16) & (lane >= sh)
        vs = [v + jnp.where(m, _take16(v, idx), 0.0) for v in vs]
    nxt = _take16(k16, jnp.minimum(lane + 1, 15))
    inr = (k16 >= cbase) & (k16 < cbase + nhalf)
    last = ((lane == 15) | (k16 != nxt)) & inr
    kidx = jnp.where(inr, k16 - cbase, 0)
    for v, off in zip(vs, offs):
        plsc.addupdate_scatter(meta_ref, [kidx + off], v, mask=last)


EPWB = EE // NS
NHALF = NPAD // 2
SLH = NHALF // NS
NACC = 5184
SUPB = 4000
NSUPB = EPWB // SUPB
NCHB = SUPB // CB
NPAIRB = NCHB // 2


def _passB_body(v_hbm, ei_hbm, ea_hbm, alpha_hbm, amax2_hbm,
                accout_hbm, meta2_hbm,
                amax, tmpa2, meta, dsts, srcs, sidxA, sidxB, sgiA, sgiB,
                ea_s, alpha_s, vrA, vrB, sbA, sbB, acc, svA, svB, ssA, ssB):
    c = lax.axis_index("c")
    s = lax.axis_index("s")
    wid = c * NS + s
    ebase = s * EPWB
    lane = _LANE()
    zero = jnp.zeros((16,), jnp.float32)
    zi = jnp.zeros((16,), jnp.int32)
    cbase = c * NHALF
    trash = NHALF + s

    pltpu.sync_copy(amax2_hbm.at[pl.ds(0, NPAD)], amax)
    for t in range(5):
        pltpu.sync_copy(amax2_hbm.at[pl.ds(NPAD + t * 2048, 2048)], tmpa2)

        def maxb(i, carry, t=t):
            off = t * 2048 + i * 16
            amax[pl.ds(off, 16)] = jnp.maximum(amax[pl.ds(off, 16)],
                                               tmpa2[pl.ds(i * 16, 16)])
            return carry

        lax.fori_loop(0, 128, maxb, 0)

    def zm(i, carry):
        meta[pl.ds(i * 16, 16)] = zero
        return carry

    lax.fori_loop(0, 3 * NHALF // 16, zm, 0)

    def zb(e, carry):
        for r in range(CC // 16):
            vrA[e, pl.ds(r * 16, 16)] = zero
        return carry

    lax.fori_loop(0, CB, zb, 0)
    for j in range(SLH // CB):
        pltpu.sync_copy(vrA, acc.at[pl.ds(s * SLH + j * CB, CB), :])
    if NACC - NHALF != 64:
        raise ValueError("trash block must be 64 rows")
    @pl.when(s == 0)
    def _zero_trash():
        pltpu.sync_copy(vrA.at[pl.ds(0, 64), :], acc.at[pl.ds(NHALF, 64), :])
    plsc.subcore_barrier()

    def zsb(e, carry):
        for r in range(CC // 16):
            sbA[e, pl.ds(r * 16, 16)] = zero
            sbB[e, pl.ds(r * 16, 16)] = zero
        return carry

    lax.fori_loop(0, CB, zsb, 0)
    tr16 = jnp.full((16,), trash, jnp.int32)
    for g in range(NG):
        sidxA[pl.ds(g * 16, 16)] = tr16
        sidxB[pl.ds(g * 16, 16)] = tr16
    pltpu.async_copy(sbA, acc.at[sidxA], ssA, add=True)
    pltpu.async_copy(sbB, acc.at[sidxB], ssB, add=True)

    def issue(ci, vr, sgi, sv):
        def cp(i, carry):
            sgi[pl.ds(i * 16, 16)] = srcs[pl.ds(ci * CB + i * 16, 16)]
            return carry

        lax.fori_loop(0, NG, cp, 0)
        pltpu.async_copy(v_hbm.at[sgi], vr, sv)

    def waitg(vr, sgi, sv):
        pltpu.make_async_copy(v_hbm.at[sgi], vr, sv).wait()

    def waits(vr, sidx, ss):
        pltpu.make_async_copy(vr, acc.at[sidx], ss).wait()

    def compute(ci, vr, sb, sidx):
        def gb(g, carry2):
            gbase = ci * CB + g * 16
            d16 = dsts[pl.ds(gbase, 16)]
            m16 = plsc.load_gather(amax, [d16])
            a16 = alpha_s[pl.ds(gbase, 16)]
            ex16 = jnp.exp(a16 - m16)
            e2 = 2 * (gbase + lane)
            ea0 = plsc.load_gather(ea_s, [e2])
            ea1 = plsc.load_gather(ea_s, [e2 + 1])
            _seg_sum_update(meta, d16, [ex16, ex16 * ea0, ex16 * ea1],
                            [0, NHALF, 2 * NHALF], cbase, NHALF)
            inr = (d16 >= cbase) & (d16 < cbase + NHALF)
            sidx[pl.ds(g * 16, 16)] = jnp.where(inr, d16 - cbase, trash)
            for e in range(16):
                xsv = _take16(ex16, zi + e)
                ea_ = g * 16 + e
                for r in range(CC // 16):
                    sb[ea_, pl.ds(r * 16, 16)] = vr[ea_, pl.ds(r * 16, 16)] * xsv
            return carry2

        lax.fori_loop(0, NG, gb, 0)

    def sup_body(sp, carry):
        base_sp = ebase + sp * SUPB
        pltpu.sync_copy(ei_hbm.at[pl.ds(EE + base_sp, SUPB)], dsts)
        pltpu.sync_copy(ei_hbm.at[pl.ds(base_sp, SUPB)], srcs)
        pltpu.sync_copy(alpha_hbm.at[pl.ds(base_sp, SUPB)], alpha_s)
        pltpu.sync_copy(ea_hbm.at[pl.ds(2 * base_sp, 2 * SUPB)], ea_s)
        issue(0, vrA, sgiA, svA)
        issue(1, vrB, sgiB, svB)

        def pair_body(p, carry2):
            cA = 2 * p
            cB = 2 * p + 1
            waitg(vrA, sgiA, svA)
            waits(sbA, sidxA, ssA)
            compute(cA, vrA, sbA, sidxA)
            pltpu.async_copy(sbA, acc.at[sidxA], ssA, add=True)
            issue(jnp.minimum(cA + 2, NCHB - 1), vrA, sgiA, svA)
            waitg(vrB, sgiB, svB)
            waits(sbB, sidxB, ssB)
            compute(cB, vrB, sbB, sidxB)
            pltpu.async_copy(sbB, acc.at[sidxB], ssB, add=True)
            issue(jnp.minimum(cB + 2, NCHB - 1), vrB, sgiB, svB)
            return carry2

        lax.fori_loop(0, NPAIRB, pair_body, 0)
        waitg(vrA, sgiA, svA)
        waitg(vrB, sgiB, svB)
        return carry

    lax.fori_loop(0, NSUPB, sup_body, 0)
    waits(sbA, sidxA, ssA)
    waits(sbB, sidxB, ssB)

    pltpu.sync_copy(meta, meta2_hbm.at[pl.ds(wid * 3 * NHALF, 3 * NHALF)])
    plsc.subcore_barrier()
    pltpu.sync_copy(acc.at[pl.ds(s * SLH, SLH), :],
                    accout_hbm.at[pl.ds(c * NHALF + s * SLH, SLH), :])


_passB = pl.kernel(
    _passB_body,
    out_type=(
        jax.ShapeDtypeStruct((NC * NHALF, CC), jnp.float32),
        jax.ShapeDtypeStruct((NW * 3 * NHALF,), jnp.float32),
    ),
    mesh=_MESH,
    compiler_params=pltpu.CompilerParams(needs_layout_passes=False),
    scratch_types=[
        pltpu.VMEM((NPAD,), jnp.float32),
        pltpu.VMEM((2048,), jnp.float32),
        pltpu.VMEM((3 * NHALF,), jnp.float32),
        pltpu.VMEM((SUPB,), jnp.int32),
        pltpu.VMEM((SUPB,), jnp.int32),
        pltpu.VMEM((CB,), jnp.int32),
        pltpu.VMEM((CB,), jnp.int32),
        pltpu.VMEM((CB,), jnp.int32),
        pltpu.VMEM((CB,), jnp.int32),
        pltpu.VMEM((2 * SUPB,), jnp.float32),
        pltpu.VMEM((SUPB,), jnp.float32),
        pltpu.VMEM((CB, CC), jnp.float32),
        pltpu.VMEM((CB, CC), jnp.float32),
        pltpu.VMEM((CB, CC), jnp.float32),
        pltpu.VMEM((CB, CC), jnp.float32),
        pltpu.VMEM_SHARED((NACC, CC), jnp.float32),
        pltpu.SemaphoreType.DMA,
        pltpu.SemaphoreType.DMA,
        pltpu.SemaphoreType.DMA,
        pltpu.SemaphoreType.DMA,
    ],
)


def _mm_kernel(act, h_ref, w_ref, b_ref, o_ref):
    y = jnp.dot(h_ref[...], w_ref[...], preferred_element_type=jnp.float32) + b_ref[...]
    if act:
        y = jnp.where(y >= 0, y, 0.01 * y)
    o_ref[...] = y


def _dense(h, W, b, act=False):
    return pl.pallas_call(
        functools.partial(_mm_kernel, act),
        out_shape=jax.ShapeDtypeStruct((h.shape[0], W.shape[1]), jnp.float32),
        grid=(10,),
        in_specs=[
            pl.BlockSpec((h.shape[0] // 10, h.shape[1]), lambda i: (i, 0)),
            pl.BlockSpec((W.shape[0], W.shape[1]), lambda i: (0, 0)),
            pl.BlockSpec((W.shape[1],), lambda i: (0,)),
        ],
        out_specs=pl.BlockSpec((h.shape[0] // 10, W.shape[1]), lambda i: (i, 0)),
    )(h, W, b)


def _dense4_kernel(h_ref, wq_ref, bq_ref, wk_ref, bk_ref, wv_ref, bv_ref,
                   ws_ref, bs_ref, oq_ref, ok_ref, ov_ref, os_ref):
    h = h_ref[...]
    oq_ref[...] = jnp.dot(h, wq_ref[...], preferred_element_type=jnp.float32) + bq_ref[...]
    ok_ref[...] = jnp.dot(h, wk_ref[...], preferred_element_type=jnp.float32) + bk_ref[...]
    ov_ref[...] = jnp.dot(h, wv_ref[...], preferred_element_type=jnp.float32) + bv_ref[...]
    os_ref[...] = jnp.dot(h, ws_ref[...], preferred_element_type=jnp.float32) + bs_ref[...]


def _dense4(h, Wq, bq, Wk, bk, Wv, bv, Ws, bs):
    wspec = pl.BlockSpec((CC, CC), lambda i: (0, 0))
    bspec = pl.BlockSpec((CC,), lambda i: (0,))
    blk = pl.BlockSpec((NN // 10, CC), lambda i: (i, 0))
    return pl.pallas_call(
        _dense4_kernel,
        out_shape=tuple(jax.ShapeDtypeStruct((NN, CC), jnp.float32)
                        for _ in range(4)),
        grid=(10,),
        in_specs=[blk, wspec, bspec, wspec, bspec, wspec, bspec, wspec, bspec],
        out_specs=(blk, blk, blk, blk),
    )(h, Wq, bq, Wk, bk, Wv, bv, Ws, bs)


def _qeT_kernel(q_ref, we_ref, o_ref):
    o_ref[...] = lax.dot_general(we_ref[...], q_ref[...],
                                 (((1,), (1,)), ((), ())),
                                 preferred_element_type=jnp.float32)


def _qeT(qs, W_e):
    return pl.pallas_call(
        _qeT_kernel,
        out_shape=jax.ShapeDtypeStruct((2, NN), jnp.float32),
    )(qs, W_e)


def _combine_kernel(accm_ref, meta2_ref, skip_ref, we_ref, o_ref):
    main = accm_ref[:NN]
    m = meta2_ref[...].reshape(NW, 3 * NHALF)
    lo = m[:NS].sum(axis=0)
    hi = m[NS:].sum(axis=0)
    den = jnp.concatenate([lo[:NHALF], hi[:NHALF]])[:NN]
    s0 = jnp.concatenate([lo[NHALF:2 * NHALF], hi[NHALF:2 * NHALF]])[:NN]
    s1 = jnp.concatenate([lo[2 * NHALF:], hi[2 * NHALF:]])[:NN]
    sw = s0[:, None] * we_ref[0][None, :] + s1[:, None] * we_ref[1][None, :]
    out = (main + sw) / (den[:, None] + 1e-16) + skip_ref[...]
    o_ref[...] = jnp.where(out >= 0, out, 0.01 * out)


def _combine(accm, meta2, skip, W_e):
    return pl.pallas_call(
        _combine_kernel,
        out_shape=jax.ShapeDtypeStruct((NN, CC), jnp.float32),
    )(accm, meta2, skip, W_e)


def kernel(x, edge_index, edge_attr, W_f, b_f, W_q, b_q, W_k, b_k, W_v, b_v,
           W_e, W_skip, b_skip):
    inv = 1.0 / jnp.sqrt(float(CC))
    Wq = W_q * inv
    bq = b_q * inv

    ei_flat = edge_index.reshape(2 * EE)
    ea_flat = edge_attr.reshape(2 * EE)
    h = _dense(x, W_f, b_f, act=True)
    for _ in range(3):
        qs, k, v, skip = _dense4(h, Wq, bq, W_k, b_k, W_v, b_v, W_skip, b_skip)
        qeT = _qeT(qs, W_e).reshape(2 * NN)
        alpha, amax2 = _passA(qs, k, qeT, ei_flat, ea_flat)
        accm, meta2 = _passB(v, ei_flat, ea_flat, alpha, amax2)
        h = _combine(accm, meta2, skip, W_e)
    return h
